# Initial kernel scaffold; baseline (speedup 1.0000x reference)
import jax
import jax.numpy as jnp
from jax.experimental import pallas as pl


def kernel(x, edge_index, edge_attr, batch, random_walk_pe, node_attr, Wl, Wr, att_w, bconv, prel, proot, pbias):
    raise NotImplementedError("write your pallas kernel here")


# speedup vs baseline: 3.7844x; 3.7844x over previous
import functools

import jax
import jax.numpy as jnp
from jax import lax
from jax.experimental import pallas as pl
from jax.experimental.pallas import tpu as pltpu
from jax.experimental.pallas import tpu_sc as plsc

N0, E, D_IN, HEADS, HID, WL = 10000, 160000, 144, 2, 128, 16
DL = 2 * HID
DACC = DL + 128
CE = 128
NSUB = 16
NCORE = 2
F32 = jnp.float32
I32 = jnp.int32


def _ceil_to(x, m):
    return m * ((x + m - 1) // m)


def _mm2_body(x_ref, wl_ref, wr_ref, xl_ref, xr_ref):
    x = x_ref[...]
    xl_ref[...] = jnp.dot(x, wl_ref[...], preferred_element_type=F32)
    xr_ref[...] = jnp.dot(x, wr_ref[...], preferred_element_type=F32)


def _tc_mm2(cur, wl, wr):
    n = cur.shape[0]
    return pl.pallas_call(
        _mm2_body,
        out_shape=(jax.ShapeDtypeStruct((n, DL), F32),
                   jax.ShapeDtypeStruct((n, DL), F32)),
    )(cur, wl, wr)


def _post_body(acc_ref, d0_ref, d1_ref, rw_ref, b_ref, prel_ref, proot_ref,
               pb_ref, hh_ref, a_ref, bsc_ref):
    acc = acc_ref[...]
    o0 = acc[:, 0:HID]
    o1 = acc[:, HID:DL]
    d0 = d0_ref[...]
    d1 = d1_ref[...]
    h = jax.nn.relu((o0 / (d0 + 1e-16) + o1 / (d1 + 1e-16)) * 0.5 + b_ref[...])
    hh_ref[:, 0:HID] = h
    hh_ref[:, HID:D_IN] = rw_ref[...]
    hh_ref[:, D_IN:DL] = jnp.zeros((h.shape[0], DL - D_IN), F32)
    a_ref[...] = jnp.dot(h, prel_ref[...], preferred_element_type=F32)
    bsc_ref[...] = jnp.dot(h, proot_ref[...], preferred_element_type=F32) + pb_ref[...]


def _tc_post(acc, den0, den1, rw, bconv, prel, proot, pb):
    n = acc.shape[0]
    return pl.pallas_call(
        _post_body,
        out_shape=(jax.ShapeDtypeStruct((n, DL), F32),
                   jax.ShapeDtypeStruct((n, 1), F32),
                   jax.ShapeDtypeStruct((n, 1), F32)),
    )(acc, den0.reshape(n, 1), den1.reshape(n, 1), rw.reshape(n, WL),
      bconv.reshape(1, HID), prel, proot, pb.reshape(1, 1))


def _finish_body(tot_ref, nxt_ref, vals_ref, tot_out_ref, cur_ref):
    sc = jnp.tanh(vals_ref[...])
    nx = nxt_ref[:, 0:HID] * sc
    mean = jnp.mean(nx, axis=0)
    mx = jnp.max(nx, axis=0)
    tot_out_ref[...] = tot_ref[...] + jnp.concatenate([mean, mx]).reshape(1, DL)
    cur_ref[:, 0:HID] = nx
    cur_ref[:, HID:D_IN] = nxt_ref[:, HID:D_IN]


def _tc_finish(tot, nxt, vals):
    kk = nxt.shape[0]
    return pl.pallas_call(
        _finish_body,
        out_shape=(jax.ShapeDtypeStruct((1, DL), F32),
                   jax.ShapeDtypeStruct((kk, D_IN), F32)),
    )(tot, nxt, vals)


def _mesh():
    return plsc.VectorSubcoreMesh(core_axis_name="c", subcore_axis_name="s")


def _iota16():
    return lax.iota(I32, 16)


_USE_SCATTER = True


def _prefix16(v, wtmp, iota):
    del iota
    x = v
    for k in (1, 2, 4, 8):
        wtmp[pl.ds(16, 16)] = x
        x = x + wtmp[pl.ds(16 - k, 16)]
    return x


def _bcast_sum(v, wtmp, iota):
    x = v
    for k in (8, 4, 2, 1):
        wtmp[pl.ds(16, 16)] = x
        left = wtmp[pl.ds(16 + k, 16)]
        right = wtmp[pl.ds(16 - k, 16)]
        x = x + jnp.where((iota & k) == 0, left, right)
    return x


def _sc_edge_attention(nh, base0, nh_p, et_pad, ce):
    cpt = et_pad // (NSUB * ce)
    nrow = nh_p + 16
    flat = nrow * DL

    @functools.partial(
        pl.kernel,
        mesh=_mesh(),
        out_type=(jax.ShapeDtypeStruct((flat,), F32),
                  jax.ShapeDtypeStruct((flat,), F32),
                  jax.ShapeDtypeStruct((nrow,), F32),
                  jax.ShapeDtypeStruct((nrow,), F32),
                  jax.ShapeDtypeStruct((nrow,), F32),
                  jax.ShapeDtypeStruct((nrow,), F32)),
        scratch_types=[
            pltpu.VMEM((ce,), I32),
            pltpu.VMEM((ce,), I32),
            pltpu.VMEM((ce,), F32),
            pltpu.VMEM((ce,), I32),
            pltpu.VMEM((ce,), F32),
            pltpu.VMEM((ce,), F32),
            pltpu.VMEM((ce, DL), F32),
            pltpu.VMEM((ce, DL), F32),
            pltpu.VMEM((2 * ce, 128), F32),
            pltpu.VMEM((2 * ce, 128), I32),
            pltpu.VMEM((DL,), F32),
            pltpu.VMEM((48,), F32),
            pltpu.VMEM((48,), I32),
            pltpu.VMEM((nrow,), F32),
            pltpu.VMEM_SHARED((flat,), F32),
            pltpu.VMEM_SHARED((nrow,), F32),
            pltpu.VMEM_SHARED((nrow,), F32),
            pltpu.SemaphoreType.DMA,
            pltpu.SemaphoreType.DMA,
            pltpu.SemaphoreType.DMA,
        ],
    )
    def k(xl_hbm, xr_hbm, s_hbm, d_hbm, m_hbm, att_hbm, zro_hbm,
          accf_hbm0, accf_hbm1, d0o_hbm0, d1o_hbm0, d0o_hbm1, d1o_hbm1,
          si_v, di_v, m_v, ri_v, w0c_v, w1c_v, xls_v, xrd_v,
          out2_v, ri2_v, att_v, wtf_v, wti_v, zd_v,
          acc_sp, den0_sp, den1_sp, sem1, sem2, sem3):
        c = lax.axis_index("c")
        s = lax.axis_index("s")
        base = base0 + c * nh
        iota = _iota16()

        pltpu.sync_copy(att_hbm, att_v)
        for j3 in range(3):
            wtf_v[pl.ds(j3 * 16, 16)] = jnp.zeros((16,), F32)
            wti_v[pl.ds(j3 * 16, 16)] = jnp.zeros((16,), I32)

        @pl.when(s == 0)
        def _():
            pltpu.sync_copy(zro_hbm, acc_sp)

            def zb(i, _):
                zd_v[pl.ds(i * 16, 16)] = jnp.zeros((16,), F32)
                return 0
            lax.fori_loop(0, nrow // 16, zb, 0)
            pltpu.sync_copy(zd_v, den0_sp)
            pltpu.sync_copy(zd_v, den1_sp)

        plsc.subcore_barrier()

        def chunk_body(j, _):
            cid = s * cpt + j
            e0 = cid * ce
            pltpu.sync_copy(s_hbm.at[pl.ds(e0, ce)], si_v)
            pltpu.sync_copy(d_hbm.at[pl.ds(e0, ce)], di_v)
            pltpu.sync_copy(m_hbm.at[pl.ds(e0, ce)], m_v)
            for v in range(ce // 16):
                d16 = di_v[pl.ds(v * 16, 16)]
                m16 = m_v[pl.ds(v * 16, 16)]
                live = (d16 >= base) & (d16 < base + nh) & (m16 > 0.0)
                trash = nh_p + (iota % 8) + (v % 2) * 8
                ri_v[pl.ds(v * 16, 16)] = jnp.where(live, d16 - base, trash)
                w0c_v[pl.ds(v * 16, 16)] = jnp.zeros((16,), F32)
                w1c_v[pl.ds(v * 16, 16)] = jnp.zeros((16,), F32)
            cp1 = pltpu.async_copy(xl_hbm.at[si_v], xls_v, sem1)
            cp2 = pltpu.async_copy(xr_hbm.at[di_v], xrd_v, sem2)
            cp1.wait()
            cp2.wait()

            def edge_body(e, _):
                xlr = xls_v.at[e]
                xrr = xrd_v.at[e]
                accs = []
                for hh in range(HEADS):
                    accv = jnp.zeros((16,), F32)
                    for f in range(HID // 16):
                        c0 = hh * HID + f * 16
                        v1 = xlr[pl.ds(c0, 16)] + xrr[pl.ds(c0, 16)]
                        v1 = jnp.maximum(v1, 0.2 * v1)
                        accv = accv + v1 * att_v[pl.ds(c0, 16)]
                    accs.append(accv)
                w0 = jnp.exp(_bcast_sum(accs[0], wtf_v, iota))
                w1 = jnp.exp(_bcast_sum(accs[1], wtf_v, iota))
                vb = (e // 16) * 16
                lane = e - vb
                oh = iota == lane
                ohf = jnp.where(oh, 1.0, 0.0)
                t0 = w0c_v[pl.ds(vb, 16)]
                w0c_v[pl.ds(vb, 16)] = t0 + w0 * ohf
                t1 = w1c_v[pl.ds(vb, 16)]
                w1c_v[pl.ds(vb, 16)] = t1 + w1 * ohf
                r16 = ri_v[pl.ds(vb, 16)]
                bmask = jnp.where(oh, r16 * DL, 0)
                bsplat = _bcast_sum(bmask, wti_v, iota)
                o0 = out2_v.at[2 * e]
                o1 = out2_v.at[2 * e + 1]
                i0 = ri2_v.at[2 * e]
                i1 = ri2_v.at[2 * e + 1]
                for f in range(HID // 16):
                    c0 = f * 16
                    o0[pl.ds(c0, 16)] = w0 * xlr[pl.ds(c0, 16)]
                    o1[pl.ds(c0, 16)] = w1 * xlr[pl.ds(HID + c0, 16)]
                    i0[pl.ds(c0, 16)] = bsplat + c0 + iota
                    i1[pl.ds(c0, 16)] = bsplat + HID + c0 + iota
                return 0

            lax.fori_loop(0, ce, edge_body, 0)
            pltpu.sync_copy(w0c_v, den0_sp.at[ri_v], add=True)
            pltpu.sync_copy(w1c_v, den1_sp.at[ri_v], add=True)
            cps = []
            for q in range(2 * ce):
                cps.append(pltpu.async_copy(
                    out2_v.at[q], acc_sp.at[ri2_v.at[q]], sem3, add=True))
            for cp in cps:
                cp.wait()
            return 0

        lax.fori_loop(0, cpt, chunk_body, 0)
        plsc.subcore_barrier()

        @pl.when((s == 0) & (c == 0))
        def _():
            pltpu.sync_copy(acc_sp, accf_hbm0)
            pltpu.sync_copy(den0_sp, d0o_hbm0)
            pltpu.sync_copy(den1_sp, d1o_hbm0)

        @pl.when((s == 0) & (c == 1))
        def _():
            pltpu.sync_copy(acc_sp, accf_hbm1)
            pltpu.sync_copy(den0_sp, d0o_hbm1)
            pltpu.sync_copy(den1_sp, d1o_hbm1)

    return k


def _sc_scorer(n_p, e4_pad):
    cpt = e4_pad // (NCORE * NSUB * CE)

    @functools.partial(
        pl.kernel,
        mesh=_mesh(),
        out_type=(jax.ShapeDtypeStruct((n_p,), F32),
                  jax.ShapeDtypeStruct((n_p,), F32)),
        scratch_types=[
            pltpu.VMEM((n_p,), F32),
            pltpu.VMEM((CE,), I32),
            pltpu.VMEM((CE,), I32),
            pltpu.VMEM((CE,), F32),
            pltpu.VMEM((CE,), F32),
            pltpu.VMEM((CE,), F32),
            pltpu.VMEM_SHARED((n_p,), F32),
            pltpu.SemaphoreType.DMA,
        ],
    )
    def k(a_hbm, s_hbm, d_hbm, m_hbm, nb0_hbm, nb1_hbm,
          z_v, si_v, di_v, m_v, ag_v, up_v, nb_sp, sem):
        c = lax.axis_index("c")
        s = lax.axis_index("s")

        @pl.when(s == 0)
        def _():
            def zb(i, _):
                z_v[pl.ds(i * 16, 16)] = jnp.zeros((16,), F32)
                return 0
            lax.fori_loop(0, n_p // 16, zb, 0)
            pltpu.sync_copy(z_v, nb_sp)

        plsc.subcore_barrier()

        def chunk_body(j, _):
            cid = (c * NSUB + s) * cpt + j
            e0 = cid * CE
            pltpu.sync_copy(s_hbm.at[pl.ds(e0, CE)], si_v)
            pltpu.sync_copy(d_hbm.at[pl.ds(e0, CE)], di_v)
            pltpu.sync_copy(m_hbm.at[pl.ds(e0, CE)], m_v)
            pltpu.async_copy(a_hbm.at[si_v], ag_v, sem).wait()
            for v in range(CE // 16):
                a16 = ag_v[pl.ds(v * 16, 16)]
                m16 = m_v[pl.ds(v * 16, 16)]
                up_v[pl.ds(v * 16, 16)] = a16 * m16
            pltpu.sync_copy(up_v, nb_sp.at[di_v], add=True)
            return 0

        lax.fori_loop(0, cpt, chunk_body, 0)
        plsc.subcore_barrier()

        @pl.when((s == 0) & (c == 0))
        def _():
            pltpu.sync_copy(nb_sp, nb0_hbm)

        @pl.when((s == 0) & (c == 1))
        def _():
            pltpu.sync_copy(nb_sp, nb1_hbm)

    return k


def _sc_select(n_p, kk):
    nv = n_p // 16

    @functools.partial(
        pl.kernel,
        mesh=_mesh(),
        out_type=(jax.ShapeDtypeStruct((n_p,), I32),
                  jax.ShapeDtypeStruct((n_p,), F32)),
        scratch_types=[
            pltpu.VMEM((n_p,), F32),
            pltpu.VMEM((n_p,), jnp.uint32),
            pltpu.VMEM((n_p,), F32),
            pltpu.VMEM((n_p,), F32),
            pltpu.VMEM((n_p,), F32),
            pltpu.VMEM((n_p,), I32),
            pltpu.VMEM((48,), I32),
        ],
    )
    def k(nb0_hbm, nb1_hbm, bsc_hbm, inv_hbm, score_hbm,
          sc_v, key_v, nb0_v, nb1_v, bsc_v, inv_v, wt_v):
        c = lax.axis_index("c")
        s = lax.axis_index("s")

        @pl.when((c == 0) & (s == 0))
        def _():
            pltpu.sync_copy(nb0_hbm, nb0_v)
            pltpu.sync_copy(nb1_hbm, nb1_v)
            pltpu.sync_copy(bsc_hbm, bsc_v)
            iota = _iota16()
            for j3 in range(3):
                wt_v[pl.ds(j3 * 16, 16)] = jnp.zeros((16,), I32)

            def keys_body(v, _):
                o = v * 16
                sc16 = (nb0_v[pl.ds(o, 16)] + nb1_v[pl.ds(o, 16)]
                        + bsc_v[pl.ds(o, 16)])
                sc_v[pl.ds(o, 16)] = sc16
                bu = lax.bitcast_convert_type(sc16, jnp.uint32)
                neg = bu >= jnp.uint32(0x80000000)
                key = jnp.where(neg, ~bu, bu | jnp.uint32(0x80000000))
                key_v[pl.ds(o, 16)] = key
                return 0

            lax.fori_loop(0, nv, keys_body, 0)

            def count_ge(tv):
                def cb(v, acc):
                    k16 = key_v[pl.ds(v * 16, 16)]
                    return acc + jnp.where(k16 >= tv, 1, 0)
                pc = lax.fori_loop(0, nv, cb, jnp.zeros((16,), I32))
                return _bcast_sum(pc, wt_v, iota)

            lo = jnp.zeros((16,), jnp.uint32)
            for bit in range(31, -1, -1):
                cand = lo | jnp.uint32(1 << bit)
                cnt = count_ge(cand)
                lo = jnp.where(cnt >= kk, cand, lo)
            vthr = lo
            c1 = count_ge(vthr + jnp.uint32(1))

            def sel_body(v, carry):
                cnt_sel, cnt_eq = carry
                o = v * 16
                k16 = key_v[pl.ds(o, 16)]
                s16 = sc_v[pl.ds(o, 16)]
                m_gt = k16 > vthr
                m_eq = k16 == vthr
                eq_i = jnp.where(m_eq, 1, 0)
                eq_pre = _prefix16(eq_i, wt_v, iota)
                eq_excl = eq_pre - eq_i
                take_eq = m_eq & ((cnt_eq + eq_excl) < (kk - c1))
                sel = m_gt | take_eq
                sel_i = jnp.where(sel, 1, 0)
                sel_pre = _prefix16(sel_i, wt_v, iota)
                rank = cnt_sel + sel_pre - sel_i
                node16 = o + iota
                del node16
                inv_v[pl.ds(o, 16)] = jnp.where(sel, rank, -1)
                return (cnt_sel + _bcast_sum(sel_i, wt_v, iota),
                        cnt_eq + _bcast_sum(eq_i, wt_v, iota))

            lax.fori_loop(0, nv, sel_body,
                          (jnp.zeros((16,), I32), jnp.zeros((16,), I32)))
            pltpu.sync_copy(inv_v, inv_hbm)
            pltpu.sync_copy(sc_v, score_hbm)

    return k


def _sc_gather_remap(n_p4, kk, k_pad, e4_pad):
    npc = n_p4 // (32 * 128)
    eb = e4_pad // 32

    @functools.partial(
        pl.kernel,
        mesh=_mesh(),
        out_type=(jax.ShapeDtypeStruct((k_pad + 16, DL), F32),
                  jax.ShapeDtypeStruct((k_pad + 16,), F32),
                  jax.ShapeDtypeStruct((e4_pad,), I32),
                  jax.ShapeDtypeStruct((e4_pad,), I32),
                  jax.ShapeDtypeStruct((e4_pad,), F32)),
        scratch_types=[
            pltpu.VMEM((128,), I32),
            pltpu.VMEM((128,), F32),
            pltpu.VMEM((128,), I32),
            pltpu.VMEM((128, DL), F32),
            pltpu.VMEM((CE,), I32),
            pltpu.VMEM((CE,), I32),
            pltpu.VMEM((CE,), F32),
            pltpu.VMEM((CE,), I32),
            pltpu.VMEM((CE,), I32),
            pltpu.VMEM((CE,), I32),
            pltpu.VMEM((CE,), I32),
            pltpu.VMEM((CE,), F32),
            pltpu.SemaphoreType.DMA,
            pltpu.SemaphoreType.DMA,
        ],
    )
    def k(hh_hbm, inv_hbm, score_hbm, s_hbm, d_hbm, m_hbm,
          nx_hbm, valsh_hbm, ns_hbm, nd_hbm, nm_hbm,
          iv_v, sv_v, ri_v, rows_v, si_v, di_v, m_v, ig_s, ig_d,
          nsv, ndv, nmv, sem1, sem2):
        c = lax.axis_index("c")
        s = lax.axis_index("s")
        t = s * NCORE + c
        iota = _iota16()

        def node_chunk(j, _):
            n0 = (t * npc + j) * 128
            pltpu.sync_copy(inv_hbm.at[pl.ds(n0, 128)], iv_v)
            pltpu.sync_copy(score_hbm.at[pl.ds(n0, 128)], sv_v)
            pltpu.sync_copy(hh_hbm.at[pl.ds(n0, 128)], rows_v)
            for v in range(8):
                iv16 = iv_v[pl.ds(v * 16, 16)]
                trash = k_pad + iota
                ri_v[pl.ds(v * 16, 16)] = jnp.where(iv16 >= 0, iv16, trash)
            pltpu.async_copy(rows_v, nx_hbm.at[ri_v], sem1).wait()
            pltpu.sync_copy(sv_v, valsh_hbm.at[ri_v])
            return 0

        lax.fori_loop(0, npc, node_chunk, 0)

        e_base = t * eb

        def chunk_body(j, _):
            e0 = e_base + j * CE
            pltpu.sync_copy(s_hbm.at[pl.ds(e0, CE)], si_v)
            pltpu.sync_copy(d_hbm.at[pl.ds(e0, CE)], di_v)
            pltpu.sync_copy(m_hbm.at[pl.ds(e0, CE)], m_v)
            cp1 = pltpu.async_copy(inv_hbm.at[si_v], ig_s, sem1)
            cp2 = pltpu.async_copy(inv_hbm.at[di_v], ig_d, sem2)
            cp1.wait()
            cp2.wait()
            for v in range(CE // 16):
                o = v * 16
                is16 = ig_s[pl.ds(o, 16)]
                id16 = ig_d[pl.ds(o, 16)]
                m16 = m_v[pl.ds(o, 16)]
                valid = (is16 >= 0) & (id16 >= 0) & (m16 > 0.0)
                sp16 = (e0 + o + iota) % kk
                nsv[pl.ds(o, 16)] = jnp.where(valid, is16, sp16)
                ndv[pl.ds(o, 16)] = jnp.where(valid, id16, sp16)
                nmv[pl.ds(o, 16)] = jnp.where(valid, 1.0, 0.0)
            pltpu.sync_copy(nsv, ns_hbm.at[pl.ds(e0, CE)])
            pltpu.sync_copy(ndv, nd_hbm.at[pl.ds(e0, CE)])
            pltpu.sync_copy(nmv, nm_hbm.at[pl.ds(e0, CE)])
            return 0

        lax.fori_loop(0, eb // CE, chunk_body, 0)

    return k


def kernel(x, edge_index, edge_attr, batch, random_walk_pe, node_attr,
           Wl, Wr, att_w, bconv, prel, proot, pbias):
    del edge_attr, batch, node_attr
    e4_pad = _ceil_to(E, NCORE * NSUB * CE)

    src = edge_index[0]
    dst = edge_index[1]
    pad4 = e4_pad - E
    sp4 = (jnp.arange(pad4, dtype=I32) % jnp.int32(N0))
    s4 = jnp.concatenate([src, sp4])
    d4 = jnp.concatenate([dst, sp4])
    m4 = jnp.concatenate([jnp.ones((E,), F32), jnp.zeros((pad4,), F32)])

    cur = x
    rw = random_walk_pe
    tot = jnp.zeros((1, DL), F32)
    n = N0
    for i in range(4):
        kk = n // 2
        ncalls = 2 if i == 0 else 1
        nh = n // (2 * ncalls)
        nh_p = _ceil_to(nh, 128)
        n_p = _ceil_to(n, 128)
        ce = 32 if nh_p > 1536 else 64
        k_pad = _ceil_to(kk, 8)

        xl, xr = _tc_mm2(cur, Wl[i], Wr[i])

        et = E + n
        et_pad = _ceil_to(et, NSUB * ce)
        pad2 = et_pad - et
        sl = jnp.arange(n, dtype=I32)
        sp2 = jnp.arange(pad2, dtype=I32) % jnp.int32(n)
        s2 = jnp.concatenate([s4[:E], sl, sp2])
        d2 = jnp.concatenate([d4[:E], sl, sp2])
        m2 = jnp.concatenate([m4[:E], jnp.ones((n,), F32),
                              jnp.zeros((pad2,), F32)])
        att_flat = att_w[i].reshape(DL)
        zro = jnp.zeros(((nh_p + 16) * DL,), F32)
        a_pieces, d0_pieces, d1_pieces = [], [], []
        for call in range(ncalls):
            af0, af1, d00, d10, d01, d11 = _sc_edge_attention(
                nh, call * 2 * nh, nh_p, et_pad, ce)(
                xl, xr, s2, d2, m2, att_flat, zro)
            a_pieces += [af0.reshape(nh_p + 16, DL)[:nh],
                         af1.reshape(nh_p + 16, DL)[:nh]]
            d0_pieces += [d00[:nh], d01[:nh]]
            d1_pieces += [d10[:nh], d11[:nh]]
        acc = jnp.concatenate(a_pieces, axis=0)
        den0 = jnp.concatenate(d0_pieces)
        den1 = jnp.concatenate(d1_pieces)

        hh, a, bsc = _tc_post(acc, den0, den1, rw, bconv[i], prel[i],
                              proot[i], pbias[i])

        a_pad = jnp.concatenate([a.reshape(n), jnp.zeros((n_p - n,), F32)])
        nb0, nb1 = _sc_scorer(n_p, e4_pad)(a_pad, s4, d4, m4)

        bsc_pad = jnp.concatenate([bsc.reshape(n),
                                   jnp.full((n_p - n,), -1e30, F32)])
        inv, score = _sc_select(n_p, kk)(nb0, nb1, bsc_pad)

        n_p4 = _ceil_to(n_p, 32 * 128)
        hh4 = jnp.concatenate([hh, jnp.zeros((n_p4 - n, DL), F32)], axis=0)
        inv4 = jnp.concatenate([inv, jnp.full((n_p4 - n_p,), -1, I32)])
        score4 = jnp.concatenate([score, jnp.zeros((n_p4 - n_p,), F32)])
        nxt, valsh, ns, nd, nm = _sc_gather_remap(n_p4, kk, k_pad, e4_pad)(
            hh4, inv4, score4, s4, d4, m4)

        tot, cur = _tc_finish(tot, nxt[:kk], valsh[:kk].reshape(kk, 1))

        rw = cur[:, HID:HID + WL]
        s4, d4, m4 = ns, nd, nm
        n = kk

    return tot


# speedup vs baseline: 5.1668x; 1.3653x over previous
import functools

import jax
import jax.numpy as jnp
from jax import lax
from jax.experimental import pallas as pl
from jax.experimental.pallas import tpu as pltpu
from jax.experimental.pallas import tpu_sc as plsc

N0, E, D_IN, HEADS, HID, WL = 10000, 160000, 144, 2, 128, 16
DL = 2 * HID
DACC = DL + 128
CE = 128
NSUB = 16
NCORE = 2
F32 = jnp.float32
I32 = jnp.int32


def _ceil_to(x, m):
    return m * ((x + m - 1) // m)


def _mm2_body(x_ref, wl_ref, wr_ref, xl_ref, xr_ref):
    x = x_ref[...]
    xl_ref[...] = jnp.dot(x, wl_ref[...], preferred_element_type=F32)
    xr_ref[...] = jnp.dot(x, wr_ref[...], preferred_element_type=F32)


def _tc_mm2(cur, wl, wr):
    n = cur.shape[0]
    return pl.pallas_call(
        _mm2_body,
        out_shape=(jax.ShapeDtypeStruct((n, DL), F32),
                   jax.ShapeDtypeStruct((n, DL), F32)),
    )(cur, wl, wr)


def _post_body(acc_ref, d0_ref, d1_ref, rw_ref, b_ref, prel_ref, proot_ref,
               pb_ref, hh_ref, a_ref, bsc_ref):
    acc = acc_ref[...]
    o0 = acc[:, 0:HID]
    o1 = acc[:, HID:DL]
    d0 = d0_ref[...]
    d1 = d1_ref[...]
    h = jax.nn.relu((o0 / (d0 + 1e-16) + o1 / (d1 + 1e-16)) * 0.5 + b_ref[...])
    hh_ref[:, 0:HID] = h
    hh_ref[:, HID:D_IN] = rw_ref[...]
    hh_ref[:, D_IN:DL] = jnp.zeros((h.shape[0], DL - D_IN), F32)
    a_ref[...] = jnp.dot(h, prel_ref[...], preferred_element_type=F32)
    bsc_ref[...] = jnp.dot(h, proot_ref[...], preferred_element_type=F32) + pb_ref[...]


def _tc_post(acc, den0, den1, rw, bconv, prel, proot, pb):
    n = acc.shape[0]
    return pl.pallas_call(
        _post_body,
        out_shape=(jax.ShapeDtypeStruct((n, DL), F32),
                   jax.ShapeDtypeStruct((n, 1), F32),
                   jax.ShapeDtypeStruct((n, 1), F32)),
    )(acc, den0.reshape(n, 1), den1.reshape(n, 1), rw.reshape(n, WL),
      bconv.reshape(1, HID), prel, proot, pb.reshape(1, 1))


def _finish_body(tot_ref, nxt_ref, vals_ref, tot_out_ref, cur_ref):
    sc = jnp.tanh(vals_ref[...])
    nx = nxt_ref[:, 0:HID] * sc
    mean = jnp.mean(nx, axis=0)
    mx = jnp.max(nx, axis=0)
    tot_out_ref[...] = tot_ref[...] + jnp.concatenate([mean, mx]).reshape(1, DL)
    cur_ref[:, 0:HID] = nx
    cur_ref[:, HID:D_IN] = nxt_ref[:, HID:D_IN]


def _tc_finish(tot, nxt, vals):
    kk = nxt.shape[0]
    return pl.pallas_call(
        _finish_body,
        out_shape=(jax.ShapeDtypeStruct((1, DL), F32),
                   jax.ShapeDtypeStruct((kk, D_IN), F32)),
    )(tot, nxt, vals)


def _mesh():
    return plsc.VectorSubcoreMesh(core_axis_name="c", subcore_axis_name="s")


def _iota16():
    return lax.iota(I32, 16)


_USE_SCATTER = True


def _prefix16(v, wtmp, iota):
    del iota
    x = v
    for k in (1, 2, 4, 8):
        wtmp[pl.ds(16, 16)] = x
        x = x + wtmp[pl.ds(16 - k, 16)]
    return x


def _bcast_sum(v, wtmp, iota):
    x = v
    for k in (8, 4, 2, 1):
        wtmp[pl.ds(16, 16)] = x
        left = wtmp[pl.ds(16 + k, 16)]
        right = wtmp[pl.ds(16 - k, 16)]
        x = x + jnp.where((iota & k) == 0, left, right)
    return x


def _sc_edge_attention(nh, base0, nh_p, et_pad, ce):
    cpt = et_pad // (NSUB * ce)
    nrow = nh_p + 16
    flat = nrow * DL

    @functools.partial(
        pl.kernel,
        mesh=_mesh(),
        out_type=(jax.ShapeDtypeStruct((2 * nrow, 128), F32),
                  jax.ShapeDtypeStruct((2 * nrow, 128), F32),
                  jax.ShapeDtypeStruct((nrow,), F32),
                  jax.ShapeDtypeStruct((nrow,), F32),
                  jax.ShapeDtypeStruct((nrow,), F32),
                  jax.ShapeDtypeStruct((nrow,), F32)),
        scratch_types=[
            pltpu.VMEM((ce,), I32),
            pltpu.VMEM((ce,), I32),
            pltpu.VMEM((ce,), F32),
            pltpu.VMEM((ce,), I32),
            pltpu.VMEM((ce,), I32),
            pltpu.VMEM((ce,), I32),
            pltpu.VMEM((ce,), F32),
            pltpu.VMEM((ce,), F32),
            pltpu.VMEM((ce, DL), F32),
            pltpu.VMEM((ce, DL), F32),
            pltpu.VMEM((ce, 128), F32),
            pltpu.VMEM((ce, 128), F32),
            pltpu.VMEM((DL,), F32),
            pltpu.VMEM((48,), F32),
            pltpu.VMEM((nrow,), F32),
            pltpu.VMEM_SHARED((2 * nrow, 128), F32),
            pltpu.VMEM_SHARED((nrow,), F32),
            pltpu.VMEM_SHARED((nrow,), F32),
            pltpu.SemaphoreType.DMA,
            pltpu.SemaphoreType.DMA,
        ],
    )
    def k(xl_hbm, xr_hbm, s_hbm, d_hbm, m_hbm, att_hbm, zro_hbm,
          accf_hbm0, accf_hbm1, d0o_hbm0, d1o_hbm0, d0o_hbm1, d1o_hbm1,
          si_v, di_v, m_v, ri_v, idx0_v, idx1_v, w0c_v, w1c_v, xls_v, xrd_v,
          oe_v, oo_v, att_v, wtf_v, zd_v,
          acc_sp, den0_sp, den1_sp, sem1, sem2):
        c = lax.axis_index("c")
        s = lax.axis_index("s")
        base = base0 + c * nh
        iota = _iota16()

        pltpu.sync_copy(att_hbm, att_v)
        for j3 in range(3):
            wtf_v[pl.ds(j3 * 16, 16)] = jnp.zeros((16,), F32)

        @pl.when(s == 0)
        def _():
            pltpu.sync_copy(zro_hbm, acc_sp)

            def zb(i, _):
                zd_v[pl.ds(i * 16, 16)] = jnp.zeros((16,), F32)
                return 0
            lax.fori_loop(0, nrow // 16, zb, 0)
            pltpu.sync_copy(zd_v, den0_sp)
            pltpu.sync_copy(zd_v, den1_sp)

        plsc.subcore_barrier()

        def chunk_body(j, _):
            cid = s * cpt + j
            e0 = cid * ce
            pltpu.sync_copy(s_hbm.at[pl.ds(e0, ce)], si_v)
            pltpu.sync_copy(d_hbm.at[pl.ds(e0, ce)], di_v)
            pltpu.sync_copy(m_hbm.at[pl.ds(e0, ce)], m_v)
            for v in range(ce // 16):
                d16 = di_v[pl.ds(v * 16, 16)]
                m16 = m_v[pl.ds(v * 16, 16)]
                live = (d16 >= base) & (d16 < base + nh) & (m16 > 0.0)
                trash = nh_p + (iota % 8) + (v % 2) * 8
                r16 = jnp.where(live, d16 - base, trash)
                ri_v[pl.ds(v * 16, 16)] = r16
                idx0_v[pl.ds(v * 16, 16)] = r16 * 2
                idx1_v[pl.ds(v * 16, 16)] = r16 * 2 + 1
                w0c_v[pl.ds(v * 16, 16)] = jnp.zeros((16,), F32)
                w1c_v[pl.ds(v * 16, 16)] = jnp.zeros((16,), F32)
            cp1 = pltpu.async_copy(xl_hbm.at[si_v], xls_v, sem1)
            cp2 = pltpu.async_copy(xr_hbm.at[di_v], xrd_v, sem2)
            cp1.wait()
            cp2.wait()

            def edge_body(e, _):
                xlr = xls_v.at[e]
                xrr = xrd_v.at[e]
                accs = []
                for hh in range(HEADS):
                    accv = jnp.zeros((16,), F32)
                    for f in range(HID // 16):
                        c0 = hh * HID + f * 16
                        v1 = xlr[pl.ds(c0, 16)] + xrr[pl.ds(c0, 16)]
                        v1 = jnp.maximum(v1, 0.2 * v1)
                        accv = accv + v1 * att_v[pl.ds(c0, 16)]
                    accs.append(accv)
                w0 = jnp.exp(_bcast_sum(accs[0], wtf_v, iota))
                w1 = jnp.exp(_bcast_sum(accs[1], wtf_v, iota))
                vb = (e // 16) * 16
                lane = e - vb
                oh = iota == lane
                ohf = jnp.where(oh, 1.0, 0.0)
                t0 = w0c_v[pl.ds(vb, 16)]
                w0c_v[pl.ds(vb, 16)] = t0 + w0 * ohf
                t1 = w1c_v[pl.ds(vb, 16)]
                w1c_v[pl.ds(vb, 16)] = t1 + w1 * ohf
                o0 = oe_v.at[e]
                o1 = oo_v.at[e]
                for f in range(HID // 16):
                    c0 = f * 16
                    o0[pl.ds(c0, 16)] = w0 * xlr[pl.ds(c0, 16)]
                    o1[pl.ds(c0, 16)] = w1 * xlr[pl.ds(HID + c0, 16)]
                return 0

            lax.fori_loop(0, ce, edge_body, 0)
            pltpu.sync_copy(w0c_v, den0_sp.at[ri_v], add=True)
            pltpu.sync_copy(w1c_v, den1_sp.at[ri_v], add=True)
            pltpu.sync_copy(oe_v, acc_sp.at[idx0_v], add=True)
            pltpu.sync_copy(oo_v, acc_sp.at[idx1_v], add=True)
            return 0

        lax.fori_loop(0, cpt, chunk_body, 0)
        plsc.subcore_barrier()

        @pl.when((s == 0) & (c == 0))
        def _():
            pltpu.sync_copy(acc_sp, accf_hbm0)
            pltpu.sync_copy(den0_sp, d0o_hbm0)
            pltpu.sync_copy(den1_sp, d1o_hbm0)

        @pl.when((s == 0) & (c == 1))
        def _():
            pltpu.sync_copy(acc_sp, accf_hbm1)
            pltpu.sync_copy(den0_sp, d0o_hbm1)
            pltpu.sync_copy(den1_sp, d1o_hbm1)

    return k


def _sc_scorer(n_p, e4_pad):
    cpt = e4_pad // (NCORE * NSUB * CE)

    @functools.partial(
        pl.kernel,
        mesh=_mesh(),
        out_type=(jax.ShapeDtypeStruct((n_p,), F32),
                  jax.ShapeDtypeStruct((n_p,), F32)),
        scratch_types=[
            pltpu.VMEM((n_p,), F32),
            pltpu.VMEM((CE,), I32),
            pltpu.VMEM((CE,), I32),
            pltpu.VMEM((CE,), F32),
            pltpu.VMEM((CE,), F32),
            pltpu.VMEM((CE,), F32),
            pltpu.VMEM_SHARED((n_p,), F32),
            pltpu.SemaphoreType.DMA,
        ],
    )
    def k(a_hbm, s_hbm, d_hbm, m_hbm, nb0_hbm, nb1_hbm,
          z_v, si_v, di_v, m_v, ag_v, up_v, nb_sp, sem):
        c = lax.axis_index("c")
        s = lax.axis_index("s")

        @pl.when(s == 0)
        def _():
            def zb(i, _):
                z_v[pl.ds(i * 16, 16)] = jnp.zeros((16,), F32)
                return 0
            lax.fori_loop(0, n_p // 16, zb, 0)
            pltpu.sync_copy(z_v, nb_sp)

        plsc.subcore_barrier()

        def chunk_body(j, _):
            cid = (c * NSUB + s) * cpt + j
            e0 = cid * CE
            pltpu.sync_copy(s_hbm.at[pl.ds(e0, CE)], si_v)
            pltpu.sync_copy(d_hbm.at[pl.ds(e0, CE)], di_v)
            pltpu.sync_copy(m_hbm.at[pl.ds(e0, CE)], m_v)
            pltpu.async_copy(a_hbm.at[si_v], ag_v, sem).wait()
            for v in range(CE // 16):
                a16 = ag_v[pl.ds(v * 16, 16)]
                m16 = m_v[pl.ds(v * 16, 16)]
                up_v[pl.ds(v * 16, 16)] = a16 * m16
            pltpu.sync_copy(up_v, nb_sp.at[di_v], add=True)
            return 0

        lax.fori_loop(0, cpt, chunk_body, 0)
        plsc.subcore_barrier()

        @pl.when((s == 0) & (c == 0))
        def _():
            pltpu.sync_copy(nb_sp, nb0_hbm)

        @pl.when((s == 0) & (c == 1))
        def _():
            pltpu.sync_copy(nb_sp, nb1_hbm)

    return k


def _sc_select(n_p, kk):
    nv = n_p // 16

    @functools.partial(
        pl.kernel,
        mesh=_mesh(),
        out_type=(jax.ShapeDtypeStruct((n_p,), I32),
                  jax.ShapeDtypeStruct((n_p,), F32)),
        scratch_types=[
            pltpu.VMEM((n_p,), F32),
            pltpu.VMEM((n_p,), jnp.uint32),
            pltpu.VMEM((n_p,), F32),
            pltpu.VMEM((n_p,), F32),
            pltpu.VMEM((n_p,), F32),
            pltpu.VMEM((n_p,), I32),
            pltpu.VMEM((48,), I32),
        ],
    )
    def k(nb0_hbm, nb1_hbm, bsc_hbm, inv_hbm, score_hbm,
          sc_v, key_v, nb0_v, nb1_v, bsc_v, inv_v, wt_v):
        c = lax.axis_index("c")
        s = lax.axis_index("s")

        @pl.when((c == 0) & (s == 0))
        def _():
            pltpu.sync_copy(nb0_hbm, nb0_v)
            pltpu.sync_copy(nb1_hbm, nb1_v)
            pltpu.sync_copy(bsc_hbm, bsc_v)
            iota = _iota16()
            for j3 in range(3):
                wt_v[pl.ds(j3 * 16, 16)] = jnp.zeros((16,), I32)

            def keys_body(v, _):
                o = v * 16
                sc16 = (nb0_v[pl.ds(o, 16)] + nb1_v[pl.ds(o, 16)]
                        + bsc_v[pl.ds(o, 16)])
                sc_v[pl.ds(o, 16)] = sc16
                bu = lax.bitcast_convert_type(sc16, jnp.uint32)
                neg = bu >= jnp.uint32(0x80000000)
                key = jnp.where(neg, ~bu, bu | jnp.uint32(0x80000000))
                key_v[pl.ds(o, 16)] = key
                return 0

            lax.fori_loop(0, nv, keys_body, 0)

            def count_ge(tv):
                def cb(v, acc):
                    k16 = key_v[pl.ds(v * 16, 16)]
                    return acc + jnp.where(k16 >= tv, 1, 0)
                pc = lax.fori_loop(0, nv, cb, jnp.zeros((16,), I32))
                return _bcast_sum(pc, wt_v, iota)

            lo = jnp.zeros((16,), jnp.uint32)
            for bit in range(31, -1, -1):
                cand = lo | jnp.uint32(1 << bit)
                cnt = count_ge(cand)
                lo = jnp.where(cnt >= kk, cand, lo)
            vthr = lo
            c1 = count_ge(vthr + jnp.uint32(1))

            def sel_body(v, carry):
                cnt_sel, cnt_eq = carry
                o = v * 16
                k16 = key_v[pl.ds(o, 16)]
                s16 = sc_v[pl.ds(o, 16)]
                m_gt = k16 > vthr
                m_eq = k16 == vthr
                eq_i = jnp.where(m_eq, 1, 0)
                eq_pre = _prefix16(eq_i, wt_v, iota)
                eq_excl = eq_pre - eq_i
                take_eq = m_eq & ((cnt_eq + eq_excl) < (kk - c1))
                sel = m_gt | take_eq
                sel_i = jnp.where(sel, 1, 0)
                sel_pre = _prefix16(sel_i, wt_v, iota)
                rank = cnt_sel + sel_pre - sel_i
                node16 = o + iota
                del node16
                inv_v[pl.ds(o, 16)] = jnp.where(sel, rank, -1)
                return (cnt_sel + _bcast_sum(sel_i, wt_v, iota),
                        cnt_eq + _bcast_sum(eq_i, wt_v, iota))

            lax.fori_loop(0, nv, sel_body,
                          (jnp.zeros((16,), I32), jnp.zeros((16,), I32)))
            pltpu.sync_copy(inv_v, inv_hbm)
            pltpu.sync_copy(sc_v, score_hbm)

    return k


def _sc_gather_remap(n_p4, kk, k_pad, e4_pad):
    npc = n_p4 // (32 * 128)
    eb = e4_pad // 32

    @functools.partial(
        pl.kernel,
        mesh=_mesh(),
        out_type=(jax.ShapeDtypeStruct((k_pad + 16, DL), F32),
                  jax.ShapeDtypeStruct((k_pad + 16,), F32),
                  jax.ShapeDtypeStruct((e4_pad,), I32),
                  jax.ShapeDtypeStruct((e4_pad,), I32),
                  jax.ShapeDtypeStruct((e4_pad,), F32)),
        scratch_types=[
            pltpu.VMEM((128,), I32),
            pltpu.VMEM((128,), F32),
            pltpu.VMEM((128,), I32),
            pltpu.VMEM((128, DL), F32),
            pltpu.VMEM((CE,), I32),
            pltpu.VMEM((CE,), I32),
            pltpu.VMEM((CE,), F32),
            pltpu.VMEM((CE,), I32),
            pltpu.VMEM((CE,), I32),
            pltpu.VMEM((CE,), I32),
            pltpu.VMEM((CE,), I32),
            pltpu.VMEM((CE,), F32),
            pltpu.SemaphoreType.DMA,
            pltpu.SemaphoreType.DMA,
        ],
    )
    def k(hh_hbm, inv_hbm, score_hbm, s_hbm, d_hbm, m_hbm,
          nx_hbm, valsh_hbm, ns_hbm, nd_hbm, nm_hbm,
          iv_v, sv_v, ri_v, rows_v, si_v, di_v, m_v, ig_s, ig_d,
          nsv, ndv, nmv, sem1, sem2):
        c = lax.axis_index("c")
        s = lax.axis_index("s")
        t = s * NCORE + c
        iota = _iota16()

        def node_chunk(j, _):
            n0 = (t * npc + j) * 128
            pltpu.sync_copy(inv_hbm.at[pl.ds(n0, 128)], iv_v)
            pltpu.sync_copy(score_hbm.at[pl.ds(n0, 128)], sv_v)
            pltpu.sync_copy(hh_hbm.at[pl.ds(n0, 128)], rows_v)
            for v in range(8):
                iv16 = iv_v[pl.ds(v * 16, 16)]
                trash = k_pad + iota
                ri_v[pl.ds(v * 16, 16)] = jnp.where(iv16 >= 0, iv16, trash)
            pltpu.async_copy(rows_v, nx_hbm.at[ri_v], sem1).wait()
            pltpu.sync_copy(sv_v, valsh_hbm.at[ri_v])
            return 0

        lax.fori_loop(0, npc, node_chunk, 0)

        e_base = t * eb

        def chunk_body(j, _):
            e0 = e_base + j * CE
            pltpu.sync_copy(s_hbm.at[pl.ds(e0, CE)], si_v)
            pltpu.sync_copy(d_hbm.at[pl.ds(e0, CE)], di_v)
            pltpu.sync_copy(m_hbm.at[pl.ds(e0, CE)], m_v)
            cp1 = pltpu.async_copy(inv_hbm.at[si_v], ig_s, sem1)
            cp2 = pltpu.async_copy(inv_hbm.at[di_v], ig_d, sem2)
            cp1.wait()
            cp2.wait()
            for v in range(CE // 16):
                o = v * 16
                is16 = ig_s[pl.ds(o, 16)]
                id16 = ig_d[pl.ds(o, 16)]
                m16 = m_v[pl.ds(o, 16)]
                valid = (is16 >= 0) & (id16 >= 0) & (m16 > 0.0)
                sp16 = (e0 + o + iota) % kk
                nsv[pl.ds(o, 16)] = jnp.where(valid, is16, sp16)
                ndv[pl.ds(o, 16)] = jnp.where(valid, id16, sp16)
                nmv[pl.ds(o, 16)] = jnp.where(valid, 1.0, 0.0)
            pltpu.sync_copy(nsv, ns_hbm.at[pl.ds(e0, CE)])
            pltpu.sync_copy(ndv, nd_hbm.at[pl.ds(e0, CE)])
            pltpu.sync_copy(nmv, nm_hbm.at[pl.ds(e0, CE)])
            return 0

        lax.fori_loop(0, eb // CE, chunk_body, 0)

    return k


def kernel(x, edge_index, edge_attr, batch, random_walk_pe, node_attr,
           Wl, Wr, att_w, bconv, prel, proot, pbias):
    del edge_attr, batch, node_attr
    e4_pad = _ceil_to(E, NCORE * NSUB * CE)

    src = edge_index[0]
    dst = edge_index[1]
    pad4 = e4_pad - E
    sp4 = (jnp.arange(pad4, dtype=I32) % jnp.int32(N0))
    s4 = jnp.concatenate([src, sp4])
    d4 = jnp.concatenate([dst, sp4])
    m4 = jnp.concatenate([jnp.ones((E,), F32), jnp.zeros((pad4,), F32)])

    cur = x
    rw = random_walk_pe
    tot = jnp.zeros((1, DL), F32)
    n = N0
    for i in range(4):
        kk = n // 2
        ncalls = 2 if i == 0 else 1
        nh = n // (2 * ncalls)
        nh_p = _ceil_to(nh, 128)
        n_p = _ceil_to(n, 128)
        ce = 64
        k_pad = _ceil_to(kk, 8)

        xl, xr = _tc_mm2(cur, Wl[i], Wr[i])

        et = E + n
        et_pad = _ceil_to(et, NSUB * ce)
        pad2 = et_pad - et
        sl = jnp.arange(n, dtype=I32)
        sp2 = jnp.arange(pad2, dtype=I32) % jnp.int32(n)
        s2 = jnp.concatenate([s4[:E], sl, sp2])
        d2 = jnp.concatenate([d4[:E], sl, sp2])
        m2 = jnp.concatenate([m4[:E], jnp.ones((n,), F32),
                              jnp.zeros((pad2,), F32)])
        att_flat = att_w[i].reshape(DL)
        zro = jnp.zeros((2 * (nh_p + 16), 128), F32)
        a_pieces, d0_pieces, d1_pieces = [], [], []
        for call in range(ncalls):
            af0, af1, d00, d10, d01, d11 = _sc_edge_attention(
                nh, call * 2 * nh, nh_p, et_pad, ce)(
                xl, xr, s2, d2, m2, att_flat, zro)
            a_pieces += [af0.reshape(nh_p + 16, DL)[:nh],
                         af1.reshape(nh_p + 16, DL)[:nh]]
            d0_pieces += [d00[:nh], d01[:nh]]
            d1_pieces += [d10[:nh], d11[:nh]]
        acc = jnp.concatenate(a_pieces, axis=0)
        den0 = jnp.concatenate(d0_pieces)
        den1 = jnp.concatenate(d1_pieces)

        hh, a, bsc = _tc_post(acc, den0, den1, rw, bconv[i], prel[i],
                              proot[i], pbias[i])

        a_pad = jnp.concatenate([a.reshape(n), jnp.zeros((n_p - n,), F32)])
        nb0, nb1 = _sc_scorer(n_p, e4_pad)(a_pad, s4, d4, m4)

        bsc_pad = jnp.concatenate([bsc.reshape(n),
                                   jnp.full((n_p - n,), -1e30, F32)])
        inv, score = _sc_select(n_p, kk)(nb0, nb1, bsc_pad)

        n_p4 = _ceil_to(n_p, 32 * 128)
        hh4 = jnp.concatenate([hh, jnp.zeros((n_p4 - n, DL), F32)], axis=0)
        inv4 = jnp.concatenate([inv, jnp.full((n_p4 - n_p,), -1, I32)])
        score4 = jnp.concatenate([score, jnp.zeros((n_p4 - n_p,), F32)])
        nxt, valsh, ns, nd, nm = _sc_gather_remap(n_p4, kk, k_pad, e4_pad)(
            hh4, inv4, score4, s4, d4, m4)

        tot, cur = _tc_finish(tot, nxt[:kk], valsh[:kk].reshape(kk, 1))

        rw = cur[:, HID:HID + WL]
        s4, d4, m4 = ns, nd, nm
        n = kk

    return tot


# speedup vs baseline: 5.9663x; 1.1547x over previous
import functools

import jax
import jax.numpy as jnp
from jax import lax
from jax.experimental import pallas as pl
from jax.experimental.pallas import tpu as pltpu
from jax.experimental.pallas import tpu_sc as plsc

N0, E, D_IN, HEADS, HID, WL = 10000, 160000, 144, 2, 128, 16
DL = 2 * HID
DACC = DL + 128
CE = 128
NSUB = 16
NCORE = 2
F32 = jnp.float32
I32 = jnp.int32


def _ceil_to(x, m):
    return m * ((x + m - 1) // m)


def _mm2_body(x_ref, wl_ref, wr_ref, xl_ref, xr_ref):
    x = x_ref[...]
    xl_ref[...] = jnp.dot(x, wl_ref[...], preferred_element_type=F32)
    xr_ref[...] = jnp.dot(x, wr_ref[...], preferred_element_type=F32)


def _tc_mm2(cur, wl, wr):
    n = cur.shape[0]
    return pl.pallas_call(
        _mm2_body,
        out_shape=(jax.ShapeDtypeStruct((n, DL), F32),
                   jax.ShapeDtypeStruct((n, DL), F32)),
    )(cur, wl, wr)


def _post_body(acc_ref, d0_ref, d1_ref, rw_ref, b_ref, prel_ref, proot_ref,
               pb_ref, hh_ref, a_ref, bsc_ref):
    acc = acc_ref[...]
    o0 = acc[:, 0:HID]
    o1 = acc[:, HID:DL]
    d0 = d0_ref[...]
    d1 = d1_ref[...]
    h = jax.nn.relu((o0 / (d0 + 1e-16) + o1 / (d1 + 1e-16)) * 0.5 + b_ref[...])
    hh_ref[:, 0:HID] = h
    hh_ref[:, HID:D_IN] = rw_ref[...]
    hh_ref[:, D_IN:DL] = jnp.zeros((h.shape[0], DL - D_IN), F32)
    a_ref[...] = jnp.dot(h, prel_ref[...], preferred_element_type=F32)
    bsc_ref[...] = jnp.dot(h, proot_ref[...], preferred_element_type=F32) + pb_ref[...]


def _tc_post(acc, den0, den1, rw, bconv, prel, proot, pb):
    n = acc.shape[0]
    return pl.pallas_call(
        _post_body,
        out_shape=(jax.ShapeDtypeStruct((n, DL), F32),
                   jax.ShapeDtypeStruct((n, 1), F32),
                   jax.ShapeDtypeStruct((n, 1), F32)),
    )(acc, den0.reshape(n, 1), den1.reshape(n, 1), rw.reshape(n, WL),
      bconv.reshape(1, HID), prel, proot, pb.reshape(1, 1))


def _finish_body(tot_ref, nxt_ref, vals_ref, tot_out_ref, cur_ref):
    sc = jnp.tanh(vals_ref[...])
    nx = nxt_ref[:, 0:HID] * sc
    mean = jnp.mean(nx, axis=0)
    mx = jnp.max(nx, axis=0)
    tot_out_ref[...] = tot_ref[...] + jnp.concatenate([mean, mx]).reshape(1, DL)
    cur_ref[:, 0:HID] = nx
    cur_ref[:, HID:D_IN] = nxt_ref[:, HID:D_IN]


def _tc_finish(tot, nxt, vals):
    kk = nxt.shape[0]
    return pl.pallas_call(
        _finish_body,
        out_shape=(jax.ShapeDtypeStruct((1, DL), F32),
                   jax.ShapeDtypeStruct((kk, D_IN), F32)),
    )(tot, nxt, vals)


def _mesh():
    return plsc.VectorSubcoreMesh(core_axis_name="c", subcore_axis_name="s")


def _iota16():
    return lax.iota(I32, 16)


_USE_SCATTER = True


def _prefix16(v, wtmp, iota):
    del iota
    x = v
    for k in (1, 2, 4, 8):
        wtmp[pl.ds(16, 16)] = x
        x = x + wtmp[pl.ds(16 - k, 16)]
    return x


def _bcast_sum(v, wtmp, iota):
    x = v
    for k in (8, 4, 2, 1):
        wtmp[pl.ds(16, 16)] = x
        left = wtmp[pl.ds(16 + k, 16)]
        right = wtmp[pl.ds(16 - k, 16)]
        x = x + jnp.where((iota & k) == 0, left, right)
    return x


def _sc_edge_attention(nh, base0, nh_p, et_pad, ce):
    cpt = et_pad // (NSUB * ce)
    nrow = nh_p + 16
    flat = nrow * DL

    @functools.partial(
        pl.kernel,
        mesh=_mesh(),
        out_type=(jax.ShapeDtypeStruct((2 * nrow, 128), F32),
                  jax.ShapeDtypeStruct((2 * nrow, 128), F32),
                  jax.ShapeDtypeStruct((nrow,), F32),
                  jax.ShapeDtypeStruct((nrow,), F32),
                  jax.ShapeDtypeStruct((nrow,), F32),
                  jax.ShapeDtypeStruct((nrow,), F32)),
        scratch_types=[
            pltpu.VMEM((ce,), I32),
            pltpu.VMEM((ce,), I32),
            pltpu.VMEM((ce,), F32),
            pltpu.VMEM((ce,), I32),
            pltpu.VMEM((ce,), I32),
            pltpu.VMEM((ce,), I32),
            pltpu.VMEM((ce,), F32),
            pltpu.VMEM((ce,), F32),
            pltpu.VMEM((ce, DL), F32),
            pltpu.VMEM((ce, DL), F32),
            pltpu.VMEM((ce, 128), F32),
            pltpu.VMEM((ce, 128), F32),
            pltpu.VMEM((DL,), F32),
            pltpu.VMEM((48,), F32),
            pltpu.VMEM((nrow,), F32),
            pltpu.VMEM_SHARED((2 * nrow, 128), F32),
            pltpu.VMEM_SHARED((nrow,), F32),
            pltpu.VMEM_SHARED((nrow,), F32),
            pltpu.SemaphoreType.DMA,
            pltpu.SemaphoreType.DMA,
        ],
    )
    def k(xl_hbm, xr_hbm, s_hbm, d_hbm, m_hbm, att_hbm, zro_hbm,
          accf_hbm0, accf_hbm1, d0o_hbm0, d1o_hbm0, d0o_hbm1, d1o_hbm1,
          si_v, di_v, m_v, ri_v, idx0_v, idx1_v, w0c_v, w1c_v, xls_v, xrd_v,
          oe_v, oo_v, att_v, wtf_v, zd_v,
          acc_sp, den0_sp, den1_sp, sem1, sem2):
        c = lax.axis_index("c")
        s = lax.axis_index("s")
        base = base0 + c * nh
        iota = _iota16()

        pltpu.sync_copy(att_hbm, att_v)
        for j3 in range(3):
            wtf_v[pl.ds(j3 * 16, 16)] = jnp.zeros((16,), F32)

        @pl.when(s == 0)
        def _():
            pltpu.sync_copy(zro_hbm, acc_sp)

            def zb(i, _):
                zd_v[pl.ds(i * 16, 16)] = jnp.zeros((16,), F32)
                return 0
            lax.fori_loop(0, nrow // 16, zb, 0)
            pltpu.sync_copy(zd_v, den0_sp)
            pltpu.sync_copy(zd_v, den1_sp)

        plsc.subcore_barrier()

        def chunk_body(j, _):
            cid = s * cpt + j
            e0 = cid * ce
            pltpu.sync_copy(s_hbm.at[pl.ds(e0, ce)], si_v)
            pltpu.sync_copy(d_hbm.at[pl.ds(e0, ce)], di_v)
            pltpu.sync_copy(m_hbm.at[pl.ds(e0, ce)], m_v)
            for v in range(ce // 16):
                d16 = di_v[pl.ds(v * 16, 16)]
                m16 = m_v[pl.ds(v * 16, 16)]
                live = (d16 >= base) & (d16 < base + nh) & (m16 > 0.0)
                trash = nh_p + (iota % 8) + (v % 2) * 8
                r16 = jnp.where(live, d16 - base, trash)
                ri_v[pl.ds(v * 16, 16)] = r16
                idx0_v[pl.ds(v * 16, 16)] = r16 * 2
                idx1_v[pl.ds(v * 16, 16)] = r16 * 2 + 1
                w0c_v[pl.ds(v * 16, 16)] = jnp.zeros((16,), F32)
                w1c_v[pl.ds(v * 16, 16)] = jnp.zeros((16,), F32)
            cp1 = pltpu.async_copy(xl_hbm.at[si_v], xls_v, sem1)
            cp2 = pltpu.async_copy(xr_hbm.at[di_v], xrd_v, sem2)
            cp1.wait()
            cp2.wait()

            def edge_body(e, _):
                xlr = xls_v.at[e]
                xrr = xrd_v.at[e]
                accs = []
                for hh in range(HEADS):
                    accv = jnp.zeros((16,), F32)
                    for f in range(HID // 16):
                        c0 = hh * HID + f * 16
                        v1 = xlr[pl.ds(c0, 16)] + xrr[pl.ds(c0, 16)]
                        v1 = jnp.maximum(v1, 0.2 * v1)
                        accv = accv + v1 * att_v[pl.ds(c0, 16)]
                    accs.append(accv)
                w0 = jnp.exp(_bcast_sum(accs[0], wtf_v, iota))
                w1 = jnp.exp(_bcast_sum(accs[1], wtf_v, iota))
                vb = (e // 16) * 16
                lane = e - vb
                oh = iota == lane
                ohf = jnp.where(oh, 1.0, 0.0)
                t0 = w0c_v[pl.ds(vb, 16)]
                w0c_v[pl.ds(vb, 16)] = t0 + w0 * ohf
                t1 = w1c_v[pl.ds(vb, 16)]
                w1c_v[pl.ds(vb, 16)] = t1 + w1 * ohf
                o0 = oe_v.at[e]
                o1 = oo_v.at[e]
                for f in range(HID // 16):
                    c0 = f * 16
                    o0[pl.ds(c0, 16)] = w0 * xlr[pl.ds(c0, 16)]
                    o1[pl.ds(c0, 16)] = w1 * xlr[pl.ds(HID + c0, 16)]
                return 0

            lax.fori_loop(0, ce, edge_body, 0)
            pltpu.sync_copy(w0c_v, den0_sp.at[ri_v], add=True)
            pltpu.sync_copy(w1c_v, den1_sp.at[ri_v], add=True)
            pltpu.sync_copy(oe_v, acc_sp.at[idx0_v], add=True)
            pltpu.sync_copy(oo_v, acc_sp.at[idx1_v], add=True)
            return 0

        lax.fori_loop(0, cpt, chunk_body, 0)
        plsc.subcore_barrier()

        @pl.when((s == 0) & (c == 0))
        def _():
            pltpu.sync_copy(acc_sp, accf_hbm0)
            pltpu.sync_copy(den0_sp, d0o_hbm0)
            pltpu.sync_copy(den1_sp, d1o_hbm0)

        @pl.when((s == 0) & (c == 1))
        def _():
            pltpu.sync_copy(acc_sp, accf_hbm1)
            pltpu.sync_copy(den0_sp, d0o_hbm1)
            pltpu.sync_copy(den1_sp, d1o_hbm1)

    return k


def _sc_scorer(n_p, e4_pad):
    cpt = e4_pad // (NCORE * NSUB * CE)

    @functools.partial(
        pl.kernel,
        mesh=_mesh(),
        out_type=(jax.ShapeDtypeStruct((n_p,), F32),
                  jax.ShapeDtypeStruct((n_p,), F32)),
        scratch_types=[
            pltpu.VMEM((n_p,), F32),
            pltpu.VMEM((CE,), I32),
            pltpu.VMEM((CE,), I32),
            pltpu.VMEM((CE,), F32),
            pltpu.VMEM((CE,), F32),
            pltpu.VMEM((CE,), F32),
            pltpu.VMEM_SHARED((n_p,), F32),
            pltpu.SemaphoreType.DMA,
        ],
    )
    def k(a_hbm, s_hbm, d_hbm, m_hbm, nb0_hbm, nb1_hbm,
          z_v, si_v, di_v, m_v, ag_v, up_v, nb_sp, sem):
        c = lax.axis_index("c")
        s = lax.axis_index("s")

        @pl.when(s == 0)
        def _():
            def zb(i, _):
                z_v[pl.ds(i * 16, 16)] = jnp.zeros((16,), F32)
                return 0
            lax.fori_loop(0, n_p // 16, zb, 0)
            pltpu.sync_copy(z_v, nb_sp)

        plsc.subcore_barrier()

        def chunk_body(j, _):
            cid = (c * NSUB + s) * cpt + j
            e0 = cid * CE
            pltpu.sync_copy(s_hbm.at[pl.ds(e0, CE)], si_v)
            pltpu.sync_copy(d_hbm.at[pl.ds(e0, CE)], di_v)
            pltpu.sync_copy(m_hbm.at[pl.ds(e0, CE)], m_v)
            pltpu.async_copy(a_hbm.at[si_v], ag_v, sem).wait()
            for v in range(CE // 16):
                a16 = ag_v[pl.ds(v * 16, 16)]
                m16 = m_v[pl.ds(v * 16, 16)]
                up_v[pl.ds(v * 16, 16)] = a16 * m16
            pltpu.sync_copy(up_v, nb_sp.at[di_v], add=True)
            return 0

        lax.fori_loop(0, cpt, chunk_body, 0)
        plsc.subcore_barrier()

        @pl.when((s == 0) & (c == 0))
        def _():
            pltpu.sync_copy(nb_sp, nb0_hbm)

        @pl.when((s == 0) & (c == 1))
        def _():
            pltpu.sync_copy(nb_sp, nb1_hbm)

    return k


def _sc_select(n_p, kk):
    nv = n_p // 16

    @functools.partial(
        pl.kernel,
        mesh=_mesh(),
        out_type=(jax.ShapeDtypeStruct((n_p,), I32),
                  jax.ShapeDtypeStruct((n_p,), F32)),
        scratch_types=[
            pltpu.VMEM((n_p,), F32),
            pltpu.VMEM((n_p,), jnp.uint32),
            pltpu.VMEM((n_p,), F32),
            pltpu.VMEM((n_p,), F32),
            pltpu.VMEM((n_p,), F32),
            pltpu.VMEM((n_p,), I32),
            pltpu.VMEM((48,), I32),
        ],
    )
    def k(nb0_hbm, nb1_hbm, bsc_hbm, inv_hbm, score_hbm,
          sc_v, key_v, nb0_v, nb1_v, bsc_v, inv_v, wt_v):
        c = lax.axis_index("c")
        s = lax.axis_index("s")

        @pl.when((c == 0) & (s == 0))
        def _():
            pltpu.sync_copy(nb0_hbm, nb0_v)
            pltpu.sync_copy(nb1_hbm, nb1_v)
            pltpu.sync_copy(bsc_hbm, bsc_v)
            iota = _iota16()
            for j3 in range(3):
                wt_v[pl.ds(j3 * 16, 16)] = jnp.zeros((16,), I32)

            def keys_body(v, _):
                o = v * 16
                sc16 = (nb0_v[pl.ds(o, 16)] + nb1_v[pl.ds(o, 16)]
                        + bsc_v[pl.ds(o, 16)])
                sc_v[pl.ds(o, 16)] = sc16
                bu = lax.bitcast_convert_type(sc16, jnp.uint32)
                neg = bu >= jnp.uint32(0x80000000)
                key = jnp.where(neg, ~bu, bu | jnp.uint32(0x80000000))
                key_v[pl.ds(o, 16)] = key
                return 0

            lax.fori_loop(0, nv, keys_body, 0)

            def count_ge(tv):
                def cb(v, acc):
                    k16 = key_v[pl.ds(v * 16, 16)]
                    return acc + jnp.where(k16 >= tv, 1, 0)
                pc = lax.fori_loop(0, nv, cb, jnp.zeros((16,), I32))
                return _bcast_sum(pc, wt_v, iota)

            lo = jnp.zeros((16,), jnp.uint32)
            for bit in range(31, -1, -1):
                cand = lo | jnp.uint32(1 << bit)
                cnt = count_ge(cand)
                lo = jnp.where(cnt >= kk, cand, lo)
            vthr = lo
            c1 = count_ge(vthr + jnp.uint32(1))

            def sel_body(v, carry):
                cnt_sel, cnt_eq = carry
                o = v * 16
                k16 = key_v[pl.ds(o, 16)]
                s16 = sc_v[pl.ds(o, 16)]
                m_gt = k16 > vthr
                m_eq = k16 == vthr
                eq_i = jnp.where(m_eq, 1, 0)
                eq_pre = _prefix16(eq_i, wt_v, iota)
                eq_excl = eq_pre - eq_i
                take_eq = m_eq & ((cnt_eq + eq_excl) < (kk - c1))
                sel = m_gt | take_eq
                sel_i = jnp.where(sel, 1, 0)
                sel_pre = _prefix16(sel_i, wt_v, iota)
                rank = cnt_sel + sel_pre - sel_i
                node16 = o + iota
                del node16
                inv_v[pl.ds(o, 16)] = jnp.where(sel, rank, -1)
                return (cnt_sel + _bcast_sum(sel_i, wt_v, iota),
                        cnt_eq + _bcast_sum(eq_i, wt_v, iota))

            lax.fori_loop(0, nv, sel_body,
                          (jnp.zeros((16,), I32), jnp.zeros((16,), I32)))
            pltpu.sync_copy(inv_v, inv_hbm)
            pltpu.sync_copy(sc_v, score_hbm)

    return k


def _sc_gather_remap(n_p4, kk, k_pad, e4_pad):
    npc = n_p4 // (32 * 128)
    eb = e4_pad // 32

    @functools.partial(
        pl.kernel,
        mesh=_mesh(),
        out_type=(jax.ShapeDtypeStruct((k_pad + 16, DL), F32),
                  jax.ShapeDtypeStruct((k_pad + 16,), F32),
                  jax.ShapeDtypeStruct((e4_pad,), I32),
                  jax.ShapeDtypeStruct((e4_pad,), I32),
                  jax.ShapeDtypeStruct((e4_pad,), F32)),
        scratch_types=[
            pltpu.VMEM((128,), I32),
            pltpu.VMEM((128,), F32),
            pltpu.VMEM((128,), I32),
            pltpu.VMEM((128, DL), F32),
            pltpu.VMEM((CE,), I32),
            pltpu.VMEM((CE,), I32),
            pltpu.VMEM((CE,), F32),
            pltpu.VMEM((CE,), I32),
            pltpu.VMEM((CE,), I32),
            pltpu.VMEM((CE,), I32),
            pltpu.VMEM((CE,), I32),
            pltpu.VMEM((CE,), F32),
            pltpu.SemaphoreType.DMA,
            pltpu.SemaphoreType.DMA,
        ],
    )
    def k(hh_hbm, inv_hbm, score_hbm, s_hbm, d_hbm, m_hbm,
          nx_hbm, valsh_hbm, ns_hbm, nd_hbm, nm_hbm,
          iv_v, sv_v, ri_v, rows_v, si_v, di_v, m_v, ig_s, ig_d,
          nsv, ndv, nmv, sem1, sem2):
        c = lax.axis_index("c")
        s = lax.axis_index("s")
        t = s * NCORE + c
        iota = _iota16()

        def node_chunk(j, _):
            n0 = (t * npc + j) * 128
            pltpu.sync_copy(inv_hbm.at[pl.ds(n0, 128)], iv_v)
            pltpu.sync_copy(score_hbm.at[pl.ds(n0, 128)], sv_v)
            pltpu.sync_copy(hh_hbm.at[pl.ds(n0, 128)], rows_v)
            for v in range(8):
                iv16 = iv_v[pl.ds(v * 16, 16)]
                trash = k_pad + iota
                ri_v[pl.ds(v * 16, 16)] = jnp.where(iv16 >= 0, iv16, trash)
            pltpu.async_copy(rows_v, nx_hbm.at[ri_v], sem1).wait()
            pltpu.sync_copy(sv_v, valsh_hbm.at[ri_v])
            return 0

        lax.fori_loop(0, npc, node_chunk, 0)

        e_base = t * eb

        def chunk_body(j, _):
            e0 = e_base + j * CE
            pltpu.sync_copy(s_hbm.at[pl.ds(e0, CE)], si_v)
            pltpu.sync_copy(d_hbm.at[pl.ds(e0, CE)], di_v)
            pltpu.sync_copy(m_hbm.at[pl.ds(e0, CE)], m_v)
            cp1 = pltpu.async_copy(inv_hbm.at[si_v], ig_s, sem1)
            cp2 = pltpu.async_copy(inv_hbm.at[di_v], ig_d, sem2)
            cp1.wait()
            cp2.wait()
            for v in range(CE // 16):
                o = v * 16
                is16 = ig_s[pl.ds(o, 16)]
                id16 = ig_d[pl.ds(o, 16)]
                m16 = m_v[pl.ds(o, 16)]
                valid = (is16 >= 0) & (id16 >= 0) & (m16 > 0.0)
                sp16 = (e0 + o + iota) % kk
                nsv[pl.ds(o, 16)] = jnp.where(valid, is16, sp16)
                ndv[pl.ds(o, 16)] = jnp.where(valid, id16, sp16)
                nmv[pl.ds(o, 16)] = jnp.where(valid, 1.0, 0.0)
            pltpu.sync_copy(nsv, ns_hbm.at[pl.ds(e0, CE)])
            pltpu.sync_copy(ndv, nd_hbm.at[pl.ds(e0, CE)])
            pltpu.sync_copy(nmv, nm_hbm.at[pl.ds(e0, CE)])
            return 0

        lax.fori_loop(0, eb // CE, chunk_body, 0)

    return k


def kernel(x, edge_index, edge_attr, batch, random_walk_pe, node_attr,
           Wl, Wr, att_w, bconv, prel, proot, pbias):
    del edge_attr, batch, node_attr
    e4_pad = _ceil_to(E, NCORE * NSUB * CE)

    src = edge_index[0]
    dst = edge_index[1]
    pad4 = e4_pad - E
    sp4 = (jnp.arange(pad4, dtype=I32) % jnp.int32(N0))
    s4 = jnp.concatenate([src, sp4])
    d4 = jnp.concatenate([dst, sp4])
    m4 = jnp.concatenate([jnp.ones((E,), F32), jnp.zeros((pad4,), F32)])

    cur = x
    rw = random_walk_pe
    tot = jnp.zeros((1, DL), F32)
    n = N0
    for i in range(4):
        kk = n // 2
        ncalls = 1
        nh = n // 2
        nh_p = _ceil_to(nh, 128)
        n_p = _ceil_to(n, 128)
        ce = 32 if i == 0 else 64
        k_pad = _ceil_to(kk, 8)

        xl, xr = _tc_mm2(cur, Wl[i], Wr[i])

        et = E + n
        et_pad = _ceil_to(et, NSUB * ce)
        pad2 = et_pad - et
        sl = jnp.arange(n, dtype=I32)
        sp2 = jnp.arange(pad2, dtype=I32) % jnp.int32(n)
        s2 = jnp.concatenate([s4[:E], sl, sp2])
        d2 = jnp.concatenate([d4[:E], sl, sp2])
        m2 = jnp.concatenate([m4[:E], jnp.ones((n,), F32),
                              jnp.zeros((pad2,), F32)])
        att_flat = att_w[i].reshape(DL)
        zro = jnp.zeros((2 * (nh_p + 16), 128), F32)
        a_pieces, d0_pieces, d1_pieces = [], [], []
        for call in range(ncalls):
            af0, af1, d00, d10, d01, d11 = _sc_edge_attention(
                nh, call * 2 * nh, nh_p, et_pad, ce)(
                xl, xr, s2, d2, m2, att_flat, zro)
            a_pieces += [af0.reshape(nh_p + 16, DL)[:nh],
                         af1.reshape(nh_p + 16, DL)[:nh]]
            d0_pieces += [d00[:nh], d01[:nh]]
            d1_pieces += [d10[:nh], d11[:nh]]
        acc = jnp.concatenate(a_pieces, axis=0)
        den0 = jnp.concatenate(d0_pieces)
        den1 = jnp.concatenate(d1_pieces)

        hh, a, bsc = _tc_post(acc, den0, den1, rw, bconv[i], prel[i],
                              proot[i], pbias[i])

        a_pad = jnp.concatenate([a.reshape(n), jnp.zeros((n_p - n,), F32)])
        nb0, nb1 = _sc_scorer(n_p, e4_pad)(a_pad, s4, d4, m4)

        bsc_pad = jnp.concatenate([bsc.reshape(n),
                                   jnp.full((n_p - n,), -1e30, F32)])
        inv, score = _sc_select(n_p, kk)(nb0, nb1, bsc_pad)

        n_p4 = _ceil_to(n_p, 32 * 128)
        hh4 = jnp.concatenate([hh, jnp.zeros((n_p4 - n, DL), F32)], axis=0)
        inv4 = jnp.concatenate([inv, jnp.full((n_p4 - n_p,), -1, I32)])
        score4 = jnp.concatenate([score, jnp.zeros((n_p4 - n_p,), F32)])
        nxt, valsh, ns, nd, nm = _sc_gather_remap(n_p4, kk, k_pad, e4_pad)(
            hh4, inv4, score4, s4, d4, m4)

        tot, cur = _tc_finish(tot, nxt[:kk], valsh[:kk].reshape(kk, 1))

        rw = cur[:, HID:HID + WL]
        s4, d4, m4 = ns, nd, nm
        n = kk

    return tot


# speedup vs baseline: 7.8191x; 1.3106x over previous
import functools

import jax
import jax.numpy as jnp
from jax import lax
from jax.experimental import pallas as pl
from jax.experimental.pallas import tpu as pltpu
from jax.experimental.pallas import tpu_sc as plsc

N0, E, D_IN, HEADS, HID, WL = 10000, 160000, 144, 2, 128, 16
DL = 2 * HID
DACC = DL + 128
CE = 128
NSUB = 16
NCORE = 2
F32 = jnp.float32
I32 = jnp.int32


def _ceil_to(x, m):
    return m * ((x + m - 1) // m)


def _mm2_body(x_ref, wl_ref, wr_ref, xl_ref, xr_ref):
    x = x_ref[...]
    xl_ref[...] = jnp.dot(x, wl_ref[...], preferred_element_type=F32)
    xr_ref[...] = jnp.dot(x, wr_ref[...], preferred_element_type=F32)


def _tc_mm2(cur, wl, wr):
    n = cur.shape[0]
    return pl.pallas_call(
        _mm2_body,
        out_shape=(jax.ShapeDtypeStruct((n, DL), F32),
                   jax.ShapeDtypeStruct((n, DL), F32)),
    )(cur, wl, wr)


def _post_body(acc_ref, d0_ref, d1_ref, rw_ref, b_ref, prel_ref, proot_ref,
               pb_ref, hh_ref, a_ref, bsc_ref):
    acc = acc_ref[...]
    o0 = acc[:, 0:HID]
    o1 = acc[:, HID:DL]
    d0 = d0_ref[...]
    d1 = d1_ref[...]
    h = jax.nn.relu((o0 / (d0 + 1e-16) + o1 / (d1 + 1e-16)) * 0.5 + b_ref[...])
    hh_ref[:, 0:HID] = h
    hh_ref[:, HID:D_IN] = rw_ref[...]
    hh_ref[:, D_IN:DL] = jnp.zeros((h.shape[0], DL - D_IN), F32)
    a_ref[...] = jnp.dot(h, prel_ref[...], preferred_element_type=F32)
    bsc_ref[...] = jnp.dot(h, proot_ref[...], preferred_element_type=F32) + pb_ref[...]


def _tc_post(acc, den0, den1, rw, bconv, prel, proot, pb):
    n = acc.shape[0]
    return pl.pallas_call(
        _post_body,
        out_shape=(jax.ShapeDtypeStruct((n, DL), F32),
                   jax.ShapeDtypeStruct((n, 1), F32),
                   jax.ShapeDtypeStruct((n, 1), F32)),
    )(acc, den0.reshape(n, 1), den1.reshape(n, 1), rw.reshape(n, WL),
      bconv.reshape(1, HID), prel, proot, pb.reshape(1, 1))


def _finish_body(tot_ref, nxt_ref, vals_ref, tot_out_ref, cur_ref):
    sc = jnp.tanh(vals_ref[...])
    nx = nxt_ref[:, 0:HID] * sc
    mean = jnp.mean(nx, axis=0)
    mx = jnp.max(nx, axis=0)
    tot_out_ref[...] = tot_ref[...] + jnp.concatenate([mean, mx]).reshape(1, DL)
    cur_ref[:, 0:HID] = nx
    cur_ref[:, HID:D_IN] = nxt_ref[:, HID:D_IN]


def _tc_finish(tot, nxt, vals):
    kk = nxt.shape[0]
    return pl.pallas_call(
        _finish_body,
        out_shape=(jax.ShapeDtypeStruct((1, DL), F32),
                   jax.ShapeDtypeStruct((kk, D_IN), F32)),
    )(tot, nxt, vals)


def _mesh():
    return plsc.VectorSubcoreMesh(core_axis_name="c", subcore_axis_name="s")


def _iota16():
    return lax.iota(I32, 16)


_USE_SCATTER = True


def _prefix16(v, wtmp, iota):
    del iota
    x = v
    for k in (1, 2, 4, 8):
        wtmp[pl.ds(16, 16)] = x
        x = x + wtmp[pl.ds(16 - k, 16)]
    return x


def _bcast_sum(v, wtmp, iota):
    x = v
    for k in (8, 4, 2, 1):
        wtmp[pl.ds(16, 16)] = x
        left = wtmp[pl.ds(16 + k, 16)]
        right = wtmp[pl.ds(16 - k, 16)]
        x = x + jnp.where((iota & k) == 0, left, right)
    return x


def _sc_edge_attention(nh, base0, nh_p, et_pad, ce, split_edges):
    cpt = et_pad // ((NCORE if split_edges else 1) * NSUB * ce)
    nrow = nh_p + 16
    flat = nrow * DL

    @functools.partial(
        pl.kernel,
        mesh=_mesh(),
        out_type=(jax.ShapeDtypeStruct((2 * nrow, 128), F32),
                  jax.ShapeDtypeStruct((2 * nrow, 128), F32),
                  jax.ShapeDtypeStruct((nrow,), F32),
                  jax.ShapeDtypeStruct((nrow,), F32),
                  jax.ShapeDtypeStruct((nrow,), F32),
                  jax.ShapeDtypeStruct((nrow,), F32)),
        scratch_types=[
            pltpu.VMEM((ce,), I32),
            pltpu.VMEM((ce,), I32),
            pltpu.VMEM((ce,), F32),
            pltpu.VMEM((ce,), I32),
            pltpu.VMEM((ce,), I32),
            pltpu.VMEM((ce,), I32),
            pltpu.VMEM((ce,), F32),
            pltpu.VMEM((ce,), F32),
            pltpu.VMEM((ce, DL), F32),
            pltpu.VMEM((ce, DL), F32),
            pltpu.VMEM((ce, 128), F32),
            pltpu.VMEM((ce, 128), F32),
            pltpu.VMEM((DL,), F32),
            pltpu.VMEM((48,), F32),
            pltpu.VMEM((48,), F32),
            pltpu.VMEM((nrow,), F32),
            pltpu.VMEM_SHARED((2 * nrow, 128), F32),
            pltpu.VMEM_SHARED((nrow,), F32),
            pltpu.VMEM_SHARED((nrow,), F32),
            pltpu.SemaphoreType.DMA,
            pltpu.SemaphoreType.DMA,
        ],
    )
    def k(xl_hbm, xr_hbm, s_hbm, d_hbm, m_hbm, att_hbm, zro_hbm,
          accf_hbm0, accf_hbm1, d0o_hbm0, d1o_hbm0, d0o_hbm1, d1o_hbm1,
          si_v, di_v, m_v, ri_v, idx0_v, idx1_v, w0c_v, w1c_v, xls_v, xrd_v,
          oe_v, oo_v, att_v, wtf_v, wtg_v, zd_v,
          acc_sp, den0_sp, den1_sp, sem1, sem2):
        c = lax.axis_index("c")
        s = lax.axis_index("s")
        base = 0 if split_edges else base0 + c * nh
        iota = _iota16()

        pltpu.sync_copy(att_hbm, att_v)
        for j3 in range(3):
            wtf_v[pl.ds(j3 * 16, 16)] = jnp.zeros((16,), F32)
            wtg_v[pl.ds(j3 * 16, 16)] = jnp.zeros((16,), F32)

        @pl.when(s == 0)
        def _():
            pltpu.sync_copy(zro_hbm, acc_sp)

            def zb(i, _):
                zd_v[pl.ds(i * 16, 16)] = jnp.zeros((16,), F32)
                return 0
            lax.fori_loop(0, nrow // 16, zb, 0)
            pltpu.sync_copy(zd_v, den0_sp)
            pltpu.sync_copy(zd_v, den1_sp)

        plsc.subcore_barrier()

        def chunk_body(j, _):
            if split_edges:
                cid = (c * NSUB + s) * cpt + j
            else:
                cid = s * cpt + j
            e0 = cid * ce
            pltpu.sync_copy(s_hbm.at[pl.ds(e0, ce)], si_v)
            pltpu.sync_copy(d_hbm.at[pl.ds(e0, ce)], di_v)
            pltpu.sync_copy(m_hbm.at[pl.ds(e0, ce)], m_v)
            for v in range(ce // 16):
                d16 = di_v[pl.ds(v * 16, 16)]
                m16 = m_v[pl.ds(v * 16, 16)]
                if split_edges:
                    live = m16 > 0.0
                else:
                    live = (d16 >= base) & (d16 < base + nh) & (m16 > 0.0)
                trash = nh_p + (iota % 8) + (v % 2) * 8
                r16 = jnp.where(live, d16 - base, trash)
                ri_v[pl.ds(v * 16, 16)] = r16
                idx0_v[pl.ds(v * 16, 16)] = r16 * 2
                idx1_v[pl.ds(v * 16, 16)] = r16 * 2 + 1
                w0c_v[pl.ds(v * 16, 16)] = jnp.zeros((16,), F32)
                w1c_v[pl.ds(v * 16, 16)] = jnp.zeros((16,), F32)
            cp1 = pltpu.async_copy(xl_hbm.at[si_v], xls_v, sem1)
            cp2 = pltpu.async_copy(xr_hbm.at[di_v], xrd_v, sem2)
            cp1.wait()
            cp2.wait()

            def edge_body(e, _):
                xlr = xls_v.at[e]
                xrr = xrd_v.at[e]
                accs = []
                for hh in range(HEADS):
                    accv = jnp.zeros((16,), F32)
                    for f in range(HID // 16):
                        c0 = hh * HID + f * 16
                        v1 = xlr[pl.ds(c0, 16)] + xrr[pl.ds(c0, 16)]
                        v1 = jnp.maximum(v1, 0.2 * v1)
                        accv = accv + v1 * att_v[pl.ds(c0, 16)]
                    accs.append(accv)
                w0 = jnp.exp(_bcast_sum(accs[0], wtf_v, iota))
                w1 = jnp.exp(_bcast_sum(accs[1], wtg_v, iota))
                vb = (e // 16) * 16
                lane = e - vb
                oh = iota == lane
                ohf = jnp.where(oh, 1.0, 0.0)
                t0 = w0c_v[pl.ds(vb, 16)]
                w0c_v[pl.ds(vb, 16)] = t0 + w0 * ohf
                t1 = w1c_v[pl.ds(vb, 16)]
                w1c_v[pl.ds(vb, 16)] = t1 + w1 * ohf
                o0 = oe_v.at[e]
                o1 = oo_v.at[e]
                for f in range(HID // 16):
                    c0 = f * 16
                    o0[pl.ds(c0, 16)] = w0 * xlr[pl.ds(c0, 16)]
                    o1[pl.ds(c0, 16)] = w1 * xlr[pl.ds(HID + c0, 16)]
                return 0

            lax.fori_loop(0, ce, edge_body, 0)
            pltpu.sync_copy(w0c_v, den0_sp.at[ri_v], add=True)
            pltpu.sync_copy(w1c_v, den1_sp.at[ri_v], add=True)
            pltpu.sync_copy(oe_v, acc_sp.at[idx0_v], add=True)
            pltpu.sync_copy(oo_v, acc_sp.at[idx1_v], add=True)
            return 0

        lax.fori_loop(0, cpt, chunk_body, 0)
        plsc.subcore_barrier()

        @pl.when((s == 0) & (c == 0))
        def _():
            pltpu.sync_copy(acc_sp, accf_hbm0)
            pltpu.sync_copy(den0_sp, d0o_hbm0)
            pltpu.sync_copy(den1_sp, d1o_hbm0)

        @pl.when((s == 0) & (c == 1))
        def _():
            pltpu.sync_copy(acc_sp, accf_hbm1)
            pltpu.sync_copy(den0_sp, d0o_hbm1)
            pltpu.sync_copy(den1_sp, d1o_hbm1)

    return k


def _sc_scorer(n_p, e4_pad):
    cpt = e4_pad // (NCORE * NSUB * CE)

    @functools.partial(
        pl.kernel,
        mesh=_mesh(),
        out_type=(jax.ShapeDtypeStruct((n_p,), F32),
                  jax.ShapeDtypeStruct((n_p,), F32)),
        scratch_types=[
            pltpu.VMEM((n_p,), F32),
            pltpu.VMEM((CE,), I32),
            pltpu.VMEM((CE,), I32),
            pltpu.VMEM((CE,), F32),
            pltpu.VMEM((CE,), F32),
            pltpu.VMEM((CE,), F32),
            pltpu.VMEM_SHARED((n_p,), F32),
            pltpu.SemaphoreType.DMA,
        ],
    )
    def k(a_hbm, s_hbm, d_hbm, m_hbm, nb0_hbm, nb1_hbm,
          z_v, si_v, di_v, m_v, ag_v, up_v, nb_sp, sem):
        c = lax.axis_index("c")
        s = lax.axis_index("s")

        @pl.when(s == 0)
        def _():
            def zb(i, _):
                z_v[pl.ds(i * 16, 16)] = jnp.zeros((16,), F32)
                return 0
            lax.fori_loop(0, n_p // 16, zb, 0)
            pltpu.sync_copy(z_v, nb_sp)

        plsc.subcore_barrier()

        def chunk_body(j, _):
            cid = (c * NSUB + s) * cpt + j
            e0 = cid * CE
            pltpu.sync_copy(s_hbm.at[pl.ds(e0, CE)], si_v)
            pltpu.sync_copy(d_hbm.at[pl.ds(e0, CE)], di_v)
            pltpu.sync_copy(m_hbm.at[pl.ds(e0, CE)], m_v)
            pltpu.async_copy(a_hbm.at[si_v], ag_v, sem).wait()
            for v in range(CE // 16):
                a16 = ag_v[pl.ds(v * 16, 16)]
                m16 = m_v[pl.ds(v * 16, 16)]
                up_v[pl.ds(v * 16, 16)] = a16 * m16
            pltpu.sync_copy(up_v, nb_sp.at[di_v], add=True)
            return 0

        lax.fori_loop(0, cpt, chunk_body, 0)
        plsc.subcore_barrier()

        @pl.when((s == 0) & (c == 0))
        def _():
            pltpu.sync_copy(nb_sp, nb0_hbm)

        @pl.when((s == 0) & (c == 1))
        def _():
            pltpu.sync_copy(nb_sp, nb1_hbm)

    return k


def _sc_select(n_p, kk):
    nv = n_p // 16

    @functools.partial(
        pl.kernel,
        mesh=_mesh(),
        out_type=(jax.ShapeDtypeStruct((n_p,), I32),
                  jax.ShapeDtypeStruct((n_p,), F32)),
        scratch_types=[
            pltpu.VMEM((n_p,), F32),
            pltpu.VMEM((n_p,), jnp.uint32),
            pltpu.VMEM((n_p,), F32),
            pltpu.VMEM((n_p,), F32),
            pltpu.VMEM((n_p,), F32),
            pltpu.VMEM((n_p,), I32),
            pltpu.VMEM((48,), I32),
        ],
    )
    def k(nb0_hbm, nb1_hbm, bsc_hbm, inv_hbm, score_hbm,
          sc_v, key_v, nb0_v, nb1_v, bsc_v, inv_v, wt_v):
        c = lax.axis_index("c")
        s = lax.axis_index("s")

        @pl.when((c == 0) & (s == 0))
        def _():
            pltpu.sync_copy(nb0_hbm, nb0_v)
            pltpu.sync_copy(nb1_hbm, nb1_v)
            pltpu.sync_copy(bsc_hbm, bsc_v)
            iota = _iota16()
            for j3 in range(3):
                wt_v[pl.ds(j3 * 16, 16)] = jnp.zeros((16,), I32)

            def keys_body(v, _):
                o = v * 16
                sc16 = (nb0_v[pl.ds(o, 16)] + nb1_v[pl.ds(o, 16)]
                        + bsc_v[pl.ds(o, 16)])
                sc_v[pl.ds(o, 16)] = sc16
                bu = lax.bitcast_convert_type(sc16, jnp.uint32)
                neg = bu >= jnp.uint32(0x80000000)
                key = jnp.where(neg, ~bu, bu | jnp.uint32(0x80000000))
                key_v[pl.ds(o, 16)] = key
                return 0

            lax.fori_loop(0, nv, keys_body, 0)

            def count_ge(tv):
                def cb(v, acc):
                    k16 = key_v[pl.ds(v * 16, 16)]
                    return acc + jnp.where(k16 >= tv, 1, 0)
                pc = lax.fori_loop(0, nv, cb, jnp.zeros((16,), I32))
                return _bcast_sum(pc, wt_v, iota)

            lo = jnp.zeros((16,), jnp.uint32)
            for bit in range(31, -1, -1):
                cand = lo | jnp.uint32(1 << bit)
                cnt = count_ge(cand)
                lo = jnp.where(cnt >= kk, cand, lo)
            vthr = lo
            c1 = count_ge(vthr + jnp.uint32(1))

            def sel_body(v, carry):
                cnt_sel, cnt_eq = carry
                o = v * 16
                k16 = key_v[pl.ds(o, 16)]
                s16 = sc_v[pl.ds(o, 16)]
                m_gt = k16 > vthr
                m_eq = k16 == vthr
                eq_i = jnp.where(m_eq, 1, 0)
                eq_pre = _prefix16(eq_i, wt_v, iota)
                eq_excl = eq_pre - eq_i
                take_eq = m_eq & ((cnt_eq + eq_excl) < (kk - c1))
                sel = m_gt | take_eq
                sel_i = jnp.where(sel, 1, 0)
                sel_pre = _prefix16(sel_i, wt_v, iota)
                rank = cnt_sel + sel_pre - sel_i
                node16 = o + iota
                del node16
                inv_v[pl.ds(o, 16)] = jnp.where(sel, rank, -1)
                return (cnt_sel + _bcast_sum(sel_i, wt_v, iota),
                        cnt_eq + _bcast_sum(eq_i, wt_v, iota))

            lax.fori_loop(0, nv, sel_body,
                          (jnp.zeros((16,), I32), jnp.zeros((16,), I32)))
            pltpu.sync_copy(inv_v, inv_hbm)
            pltpu.sync_copy(sc_v, score_hbm)

    return k


def _sc_gather_remap(n_p4, kk, k_pad, e4_pad):
    npc = n_p4 // (32 * 128)
    eb = e4_pad // 32

    @functools.partial(
        pl.kernel,
        mesh=_mesh(),
        out_type=(jax.ShapeDtypeStruct((k_pad + 16, DL), F32),
                  jax.ShapeDtypeStruct((k_pad + 16,), F32),
                  jax.ShapeDtypeStruct((e4_pad,), I32),
                  jax.ShapeDtypeStruct((e4_pad,), I32),
                  jax.ShapeDtypeStruct((e4_pad,), F32)),
        scratch_types=[
            pltpu.VMEM((128,), I32),
            pltpu.VMEM((128,), F32),
            pltpu.VMEM((128,), I32),
            pltpu.VMEM((128, DL), F32),
            pltpu.VMEM((CE,), I32),
            pltpu.VMEM((CE,), I32),
            pltpu.VMEM((CE,), F32),
            pltpu.VMEM((CE,), I32),
            pltpu.VMEM((CE,), I32),
            pltpu.VMEM((CE,), I32),
            pltpu.VMEM((CE,), I32),
            pltpu.VMEM((CE,), F32),
            pltpu.SemaphoreType.DMA,
            pltpu.SemaphoreType.DMA,
        ],
    )
    def k(hh_hbm, inv_hbm, score_hbm, s_hbm, d_hbm, m_hbm,
          nx_hbm, valsh_hbm, ns_hbm, nd_hbm, nm_hbm,
          iv_v, sv_v, ri_v, rows_v, si_v, di_v, m_v, ig_s, ig_d,
          nsv, ndv, nmv, sem1, sem2):
        c = lax.axis_index("c")
        s = lax.axis_index("s")
        t = s * NCORE + c
        iota = _iota16()

        def node_chunk(j, _):
            n0 = (t * npc + j) * 128
            pltpu.sync_copy(inv_hbm.at[pl.ds(n0, 128)], iv_v)
            pltpu.sync_copy(score_hbm.at[pl.ds(n0, 128)], sv_v)
            pltpu.sync_copy(hh_hbm.at[pl.ds(n0, 128)], rows_v)
            for v in range(8):
                iv16 = iv_v[pl.ds(v * 16, 16)]
                trash = k_pad + iota
                ri_v[pl.ds(v * 16, 16)] = jnp.where(iv16 >= 0, iv16, trash)
            pltpu.async_copy(rows_v, nx_hbm.at[ri_v], sem1).wait()
            pltpu.sync_copy(sv_v, valsh_hbm.at[ri_v])
            return 0

        lax.fori_loop(0, npc, node_chunk, 0)

        e_base = t * eb

        def chunk_body(j, _):
            e0 = e_base + j * CE
            pltpu.sync_copy(s_hbm.at[pl.ds(e0, CE)], si_v)
            pltpu.sync_copy(d_hbm.at[pl.ds(e0, CE)], di_v)
            pltpu.sync_copy(m_hbm.at[pl.ds(e0, CE)], m_v)
            cp1 = pltpu.async_copy(inv_hbm.at[si_v], ig_s, sem1)
            cp2 = pltpu.async_copy(inv_hbm.at[di_v], ig_d, sem2)
            cp1.wait()
            cp2.wait()
            for v in range(CE // 16):
                o = v * 16
                is16 = ig_s[pl.ds(o, 16)]
                id16 = ig_d[pl.ds(o, 16)]
                m16 = m_v[pl.ds(o, 16)]
                valid = (is16 >= 0) & (id16 >= 0) & (m16 > 0.0)
                sp16 = (e0 + o + iota) % kk
                nsv[pl.ds(o, 16)] = jnp.where(valid, is16, sp16)
                ndv[pl.ds(o, 16)] = jnp.where(valid, id16, sp16)
                nmv[pl.ds(o, 16)] = jnp.where(valid, 1.0, 0.0)
            pltpu.sync_copy(nsv, ns_hbm.at[pl.ds(e0, CE)])
            pltpu.sync_copy(ndv, nd_hbm.at[pl.ds(e0, CE)])
            pltpu.sync_copy(nmv, nm_hbm.at[pl.ds(e0, CE)])
            return 0

        lax.fori_loop(0, eb // CE, chunk_body, 0)

    return k


def kernel(x, edge_index, edge_attr, batch, random_walk_pe, node_attr,
           Wl, Wr, att_w, bconv, prel, proot, pbias):
    del edge_attr, batch, node_attr
    e4_pad = _ceil_to(E, NCORE * NSUB * CE)

    src = edge_index[0]
    dst = edge_index[1]
    pad4 = e4_pad - E
    sp4 = (jnp.arange(pad4, dtype=I32) % jnp.int32(N0))
    s4 = jnp.concatenate([src, sp4])
    d4 = jnp.concatenate([dst, sp4])
    m4 = jnp.concatenate([jnp.ones((E,), F32), jnp.zeros((pad4,), F32)])

    cur = x
    rw = random_walk_pe
    tot = jnp.zeros((1, DL), F32)
    n = N0
    for i in range(4):
        kk = n // 2
        ncalls = 1
        n_p = _ceil_to(n, 128)
        split_edges = i > 0
        nh = n if split_edges else n // 2
        nh_p = _ceil_to(nh, 128)
        ce = 32
        k_pad = _ceil_to(kk, 8)

        xl, xr = _tc_mm2(cur, Wl[i], Wr[i])

        et = E + n
        et_pad = _ceil_to(et, (NCORE if split_edges else 1) * NSUB * ce)
        pad2 = et_pad - et
        sl = jnp.arange(n, dtype=I32)
        sp2 = jnp.arange(pad2, dtype=I32) % jnp.int32(n)
        s2 = jnp.concatenate([s4[:E], sl, sp2])
        d2 = jnp.concatenate([d4[:E], sl, sp2])
        m2 = jnp.concatenate([m4[:E], jnp.ones((n,), F32),
                              jnp.zeros((pad2,), F32)])
        att_flat = att_w[i].reshape(DL)
        zro = jnp.zeros((2 * (nh_p + 16), 128), F32)
        del ncalls
        af0, af1, d00, d10, d01, d11 = _sc_edge_attention(
            nh, 0, nh_p, et_pad, ce, split_edges)(
            xl, xr, s2, d2, m2, att_flat, zro)
        if split_edges:
            acc = (af0 + af1).reshape(nh_p + 16, DL)[:n]
            den0 = (d00 + d01)[:n]
            den1 = (d10 + d11)[:n]
        else:
            acc = jnp.concatenate(
                [af0.reshape(nh_p + 16, DL)[:nh],
                 af1.reshape(nh_p + 16, DL)[:nh]], axis=0)
            den0 = jnp.concatenate([d00[:nh], d01[:nh]])
            den1 = jnp.concatenate([d10[:nh], d11[:nh]])

        hh, a, bsc = _tc_post(acc, den0, den1, rw, bconv[i], prel[i],
                              proot[i], pbias[i])

        a_pad = jnp.concatenate([a.reshape(n), jnp.zeros((n_p - n,), F32)])
        nb0, nb1 = _sc_scorer(n_p, e4_pad)(a_pad, s4, d4, m4)

        bsc_pad = jnp.concatenate([bsc.reshape(n),
                                   jnp.full((n_p - n,), -1e30, F32)])
        inv, score = _sc_select(n_p, kk)(nb0, nb1, bsc_pad)

        n_p4 = _ceil_to(n_p, 32 * 128)
        hh4 = jnp.concatenate([hh, jnp.zeros((n_p4 - n, DL), F32)], axis=0)
        inv4 = jnp.concatenate([inv, jnp.full((n_p4 - n_p,), -1, I32)])
        score4 = jnp.concatenate([score, jnp.zeros((n_p4 - n_p,), F32)])
        nxt, valsh, ns, nd, nm = _sc_gather_remap(n_p4, kk, k_pad, e4_pad)(
            hh4, inv4, score4, s4, d4, m4)

        tot, cur = _tc_finish(tot, nxt[:kk], valsh[:kk].reshape(kk, 1))

        rw = cur[:, HID:HID + WL]
        s4, d4, m4 = ns, nd, nm
        n = kk

    return tot


# speedup vs baseline: 8.4670x; 1.0829x over previous
import functools

import jax
import jax.numpy as jnp
from jax import lax
from jax.experimental import pallas as pl
from jax.experimental.pallas import tpu as pltpu
from jax.experimental.pallas import tpu_sc as plsc

N0, E, D_IN, HEADS, HID, WL = 10000, 160000, 144, 2, 128, 16
DL = 2 * HID
DACC = DL + 128
CE = 128
NSUB = 16
NCORE = 2
F32 = jnp.float32
I32 = jnp.int32


def _ceil_to(x, m):
    return m * ((x + m - 1) // m)


def _mm2_body(x_ref, wl_ref, wr_ref, xl_ref, xr_ref):
    x = x_ref[...]
    xl_ref[...] = jnp.dot(x, wl_ref[...], preferred_element_type=F32)
    xr_ref[...] = jnp.dot(x, wr_ref[...], preferred_element_type=F32)


def _tc_mm2(cur, wl, wr):
    n = cur.shape[0]
    return pl.pallas_call(
        _mm2_body,
        out_shape=(jax.ShapeDtypeStruct((n, DL), F32),
                   jax.ShapeDtypeStruct((n, DL), F32)),
    )(cur, wl, wr)


def _post_body(acc_ref, d0_ref, d1_ref, rw_ref, b_ref, prel_ref, proot_ref,
               pb_ref, hh_ref, a_ref, bsc_ref):
    acc = acc_ref[...]
    o0 = acc[:, 0:HID]
    o1 = acc[:, HID:DL]
    d0 = d0_ref[...]
    d1 = d1_ref[...]
    h = jax.nn.relu((o0 / (d0 + 1e-16) + o1 / (d1 + 1e-16)) * 0.5 + b_ref[...])
    hh_ref[:, 0:HID] = h
    hh_ref[:, HID:D_IN] = rw_ref[...]
    hh_ref[:, D_IN:DL] = jnp.zeros((h.shape[0], DL - D_IN), F32)
    a_ref[...] = jnp.dot(h, prel_ref[...], preferred_element_type=F32)
    bsc_ref[...] = jnp.dot(h, proot_ref[...], preferred_element_type=F32) + pb_ref[...]


def _tc_post(acc, den0, den1, rw, bconv, prel, proot, pb):
    n = acc.shape[0]
    return pl.pallas_call(
        _post_body,
        out_shape=(jax.ShapeDtypeStruct((n, DL), F32),
                   jax.ShapeDtypeStruct((n, 1), F32),
                   jax.ShapeDtypeStruct((n, 1), F32)),
    )(acc, den0.reshape(n, 1), den1.reshape(n, 1), rw.reshape(n, WL),
      bconv.reshape(1, HID), prel, proot, pb.reshape(1, 1))


def _finish_body(tot_ref, nxt_ref, vals_ref, tot_out_ref, cur_ref):
    sc = jnp.tanh(vals_ref[...])
    nx = nxt_ref[:, 0:HID] * sc
    mean = jnp.mean(nx, axis=0)
    mx = jnp.max(nx, axis=0)
    tot_out_ref[...] = tot_ref[...] + jnp.concatenate([mean, mx]).reshape(1, DL)
    cur_ref[:, 0:HID] = nx
    cur_ref[:, HID:D_IN] = nxt_ref[:, HID:D_IN]


def _tc_finish(tot, nxt, vals):
    kk = nxt.shape[0]
    return pl.pallas_call(
        _finish_body,
        out_shape=(jax.ShapeDtypeStruct((1, DL), F32),
                   jax.ShapeDtypeStruct((kk, D_IN), F32)),
    )(tot, nxt, vals)


def _mesh():
    return plsc.VectorSubcoreMesh(core_axis_name="c", subcore_axis_name="s")


def _iota16():
    return lax.iota(I32, 16)


_USE_SCATTER = True


def _prefix16(v, wtmp, iota):
    del iota
    x = v
    for k in (1, 2, 4, 8):
        wtmp[pl.ds(16, 16)] = x
        x = x + wtmp[pl.ds(16 - k, 16)]
    return x


def _bcast_sum(v, wtmp, iota):
    x = v
    for k in (8, 4, 2, 1):
        wtmp[pl.ds(16, 16)] = x
        left = wtmp[pl.ds(16 + k, 16)]
        right = wtmp[pl.ds(16 - k, 16)]
        x = x + jnp.where((iota & k) == 0, left, right)
    return x


def _sc_edge_attention(nh, base0, nh_p, et_pad, ce, split_edges):
    cpt = et_pad // ((NCORE if split_edges else 1) * NSUB * ce)
    nrow = nh_p + 16
    flat = nrow * DL

    @functools.partial(
        pl.kernel,
        mesh=_mesh(),
        out_type=(jax.ShapeDtypeStruct((2 * nrow, 128), F32),
                  jax.ShapeDtypeStruct((2 * nrow, 128), F32),
                  jax.ShapeDtypeStruct((nrow,), F32),
                  jax.ShapeDtypeStruct((nrow,), F32),
                  jax.ShapeDtypeStruct((nrow,), F32),
                  jax.ShapeDtypeStruct((nrow,), F32)),
        scratch_types=[
            pltpu.VMEM((ce,), I32),
            pltpu.VMEM((ce,), I32),
            pltpu.VMEM((ce,), F32),
            pltpu.VMEM((ce,), I32),
            pltpu.VMEM((ce,), I32),
            pltpu.VMEM((ce,), I32),
            pltpu.VMEM((ce,), F32),
            pltpu.VMEM((ce,), F32),
            pltpu.VMEM((ce, DL), F32),
            pltpu.VMEM((ce, DL), F32),
            pltpu.VMEM((ce, 128), F32),
            pltpu.VMEM((ce, 128), F32),
            pltpu.VMEM((DL,), F32),
            pltpu.VMEM((48,), F32),
            pltpu.VMEM((48,), F32),
            pltpu.VMEM((nrow,), F32),
            pltpu.VMEM_SHARED((2 * nrow, 128), F32),
            pltpu.VMEM_SHARED((nrow,), F32),
            pltpu.VMEM_SHARED((nrow,), F32),
            pltpu.SemaphoreType.DMA,
            pltpu.SemaphoreType.DMA,
        ],
    )
    def k(xl_hbm, xr_hbm, s_hbm, d_hbm, m_hbm, att_hbm, zro_hbm,
          accf_hbm0, accf_hbm1, d0o_hbm0, d1o_hbm0, d0o_hbm1, d1o_hbm1,
          si_v, di_v, m_v, ri_v, idx0_v, idx1_v, w0c_v, w1c_v, xls_v, xrd_v,
          oe_v, oo_v, att_v, wtf_v, wtg_v, zd_v,
          acc_sp, den0_sp, den1_sp, sem1, sem2):
        c = lax.axis_index("c")
        s = lax.axis_index("s")
        base = 0 if split_edges else base0 + c * nh
        iota = _iota16()

        pltpu.sync_copy(att_hbm, att_v)
        for j3 in range(3):
            wtf_v[pl.ds(j3 * 16, 16)] = jnp.zeros((16,), F32)
            wtg_v[pl.ds(j3 * 16, 16)] = jnp.zeros((16,), F32)

        @pl.when(s == 0)
        def _():
            pltpu.sync_copy(zro_hbm, acc_sp)

            def zb(i, _):
                zd_v[pl.ds(i * 16, 16)] = jnp.zeros((16,), F32)
                return 0
            lax.fori_loop(0, nrow // 16, zb, 0)
            pltpu.sync_copy(zd_v, den0_sp)
            pltpu.sync_copy(zd_v, den1_sp)

        plsc.subcore_barrier()

        def chunk_body(j, _):
            if split_edges:
                cid = (c * NSUB + s) * cpt + j
            else:
                cid = s * cpt + j
            e0 = cid * ce
            cpa = pltpu.async_copy(s_hbm.at[pl.ds(e0, ce)], si_v, sem1)
            cpb = pltpu.async_copy(d_hbm.at[pl.ds(e0, ce)], di_v, sem2)
            pltpu.sync_copy(m_hbm.at[pl.ds(e0, ce)], m_v)
            cpa.wait()
            cpb.wait()
            for v in range(ce // 16):
                d16 = di_v[pl.ds(v * 16, 16)]
                m16 = m_v[pl.ds(v * 16, 16)]
                if split_edges:
                    live = m16 > 0.0
                else:
                    live = (d16 >= base) & (d16 < base + nh) & (m16 > 0.0)
                trash = nh_p + (iota % 8) + (v % 2) * 8
                r16 = jnp.where(live, d16 - base, trash)
                ri_v[pl.ds(v * 16, 16)] = r16
                idx0_v[pl.ds(v * 16, 16)] = r16 * 2
                idx1_v[pl.ds(v * 16, 16)] = r16 * 2 + 1
                w0c_v[pl.ds(v * 16, 16)] = jnp.zeros((16,), F32)
                w1c_v[pl.ds(v * 16, 16)] = jnp.zeros((16,), F32)
            cp1 = pltpu.async_copy(xl_hbm.at[si_v], xls_v, sem1)
            cp2 = pltpu.async_copy(xr_hbm.at[di_v], xrd_v, sem2)
            cp1.wait()
            cp2.wait()

            def edge_body(e, _):
                xlr = xls_v.at[e]
                xrr = xrd_v.at[e]
                accs = []
                for hh in range(HEADS):
                    accv = jnp.zeros((16,), F32)
                    for f in range(HID // 16):
                        c0 = hh * HID + f * 16
                        v1 = xlr[pl.ds(c0, 16)] + xrr[pl.ds(c0, 16)]
                        v1 = jnp.maximum(v1, 0.2 * v1)
                        accv = accv + v1 * att_v[pl.ds(c0, 16)]
                    accs.append(accv)
                w0 = jnp.exp(_bcast_sum(accs[0], wtf_v, iota))
                w1 = jnp.exp(_bcast_sum(accs[1], wtg_v, iota))
                vb = (e // 16) * 16
                lane = e - vb
                oh = iota == lane
                ohf = jnp.where(oh, 1.0, 0.0)
                t0 = w0c_v[pl.ds(vb, 16)]
                w0c_v[pl.ds(vb, 16)] = t0 + w0 * ohf
                t1 = w1c_v[pl.ds(vb, 16)]
                w1c_v[pl.ds(vb, 16)] = t1 + w1 * ohf
                o0 = oe_v.at[e]
                o1 = oo_v.at[e]
                for f in range(HID // 16):
                    c0 = f * 16
                    o0[pl.ds(c0, 16)] = w0 * xlr[pl.ds(c0, 16)]
                    o1[pl.ds(c0, 16)] = w1 * xlr[pl.ds(HID + c0, 16)]
                return 0

            lax.fori_loop(0, ce, edge_body, 0)
            pltpu.sync_copy(w0c_v, den0_sp.at[ri_v], add=True)
            pltpu.sync_copy(w1c_v, den1_sp.at[ri_v], add=True)
            pltpu.sync_copy(oe_v, acc_sp.at[idx0_v], add=True)
            pltpu.sync_copy(oo_v, acc_sp.at[idx1_v], add=True)
            return 0

        lax.fori_loop(0, cpt, chunk_body, 0)
        plsc.subcore_barrier()

        @pl.when((s == 0) & (c == 0))
        def _():
            pltpu.sync_copy(acc_sp, accf_hbm0)
            pltpu.sync_copy(den0_sp, d0o_hbm0)
            pltpu.sync_copy(den1_sp, d1o_hbm0)

        @pl.when((s == 0) & (c == 1))
        def _():
            pltpu.sync_copy(acc_sp, accf_hbm1)
            pltpu.sync_copy(den0_sp, d0o_hbm1)
            pltpu.sync_copy(den1_sp, d1o_hbm1)

    return k


def _sc_scorer(n_p, e4_pad):
    cpt = e4_pad // (NCORE * NSUB * CE)

    @functools.partial(
        pl.kernel,
        mesh=_mesh(),
        out_type=(jax.ShapeDtypeStruct((n_p,), F32),
                  jax.ShapeDtypeStruct((n_p,), F32)),
        scratch_types=[
            pltpu.VMEM((n_p,), F32),
            pltpu.VMEM((CE,), I32),
            pltpu.VMEM((CE,), I32),
            pltpu.VMEM((CE,), F32),
            pltpu.VMEM((CE,), F32),
            pltpu.VMEM((CE,), F32),
            pltpu.VMEM_SHARED((n_p,), F32),
            pltpu.SemaphoreType.DMA,
            pltpu.SemaphoreType.DMA,
            pltpu.SemaphoreType.DMA,
        ],
    )
    def k(a_hbm, s_hbm, d_hbm, m_hbm, nb0_hbm, nb1_hbm,
          z_v, si_v, di_v, m_v, ag_v, up_v, nb_sp, sem, sem2, sem3):
        c = lax.axis_index("c")
        s = lax.axis_index("s")

        @pl.when(s == 0)
        def _():
            def zb(i, _):
                z_v[pl.ds(i * 16, 16)] = jnp.zeros((16,), F32)
                return 0
            lax.fori_loop(0, n_p // 16, zb, 0)
            pltpu.sync_copy(z_v, nb_sp)

        plsc.subcore_barrier()

        def chunk_body(j, _):
            cid = (c * NSUB + s) * cpt + j
            e0 = cid * CE
            cp1 = pltpu.async_copy(s_hbm.at[pl.ds(e0, CE)], si_v, sem)
            cp2 = pltpu.async_copy(d_hbm.at[pl.ds(e0, CE)], di_v, sem2)
            cp3 = pltpu.async_copy(m_hbm.at[pl.ds(e0, CE)], m_v, sem3)
            cp1.wait()
            cpg = pltpu.async_copy(a_hbm.at[si_v], ag_v, sem)
            cp2.wait()
            cp3.wait()
            cpg.wait()
            for v in range(CE // 16):
                a16 = ag_v[pl.ds(v * 16, 16)]
                m16 = m_v[pl.ds(v * 16, 16)]
                up_v[pl.ds(v * 16, 16)] = a16 * m16
            pltpu.sync_copy(up_v, nb_sp.at[di_v], add=True)
            return 0

        lax.fori_loop(0, cpt, chunk_body, 0)
        plsc.subcore_barrier()

        @pl.when((s == 0) & (c == 0))
        def _():
            pltpu.sync_copy(nb_sp, nb0_hbm)

        @pl.when((s == 0) & (c == 1))
        def _():
            pltpu.sync_copy(nb_sp, nb1_hbm)

    return k


def _sc_select(n_p, kk):
    nv = n_p // 16

    @functools.partial(
        pl.kernel,
        mesh=_mesh(),
        out_type=(jax.ShapeDtypeStruct((n_p,), I32),
                  jax.ShapeDtypeStruct((n_p,), F32)),
        scratch_types=[
            pltpu.VMEM((n_p,), F32),
            pltpu.VMEM((n_p,), jnp.uint32),
            pltpu.VMEM((n_p,), F32),
            pltpu.VMEM((n_p,), F32),
            pltpu.VMEM((n_p,), F32),
            pltpu.VMEM((n_p,), I32),
            pltpu.VMEM((48,), I32),
        ],
    )
    def k(nb0_hbm, nb1_hbm, bsc_hbm, inv_hbm, score_hbm,
          sc_v, key_v, nb0_v, nb1_v, bsc_v, inv_v, wt_v):
        c = lax.axis_index("c")
        s = lax.axis_index("s")

        @pl.when((c == 0) & (s == 0))
        def _():
            pltpu.sync_copy(nb0_hbm, nb0_v)
            pltpu.sync_copy(nb1_hbm, nb1_v)
            pltpu.sync_copy(bsc_hbm, bsc_v)
            iota = _iota16()
            for j3 in range(3):
                wt_v[pl.ds(j3 * 16, 16)] = jnp.zeros((16,), I32)

            def keys_body(v, _):
                o = v * 16
                sc16 = (nb0_v[pl.ds(o, 16)] + nb1_v[pl.ds(o, 16)]
                        + bsc_v[pl.ds(o, 16)])
                sc_v[pl.ds(o, 16)] = sc16
                bu = lax.bitcast_convert_type(sc16, jnp.uint32)
                neg = bu >= jnp.uint32(0x80000000)
                key = jnp.where(neg, ~bu, bu | jnp.uint32(0x80000000))
                key_v[pl.ds(o, 16)] = key
                return 0

            lax.fori_loop(0, nv, keys_body, 0)

            def count_ge(tv):
                def cb(v, acc):
                    k16 = key_v[pl.ds(v * 16, 16)]
                    return acc + jnp.where(k16 >= tv, 1, 0)
                pc = lax.fori_loop(0, nv, cb, jnp.zeros((16,), I32))
                return _bcast_sum(pc, wt_v, iota)

            lo = jnp.zeros((16,), jnp.uint32)
            for bit in range(31, -1, -1):
                cand = lo | jnp.uint32(1 << bit)
                cnt = count_ge(cand)
                lo = jnp.where(cnt >= kk, cand, lo)
            vthr = lo
            c1 = count_ge(vthr + jnp.uint32(1))

            def sel_body(v, carry):
                cnt_sel, cnt_eq = carry
                o = v * 16
                k16 = key_v[pl.ds(o, 16)]
                s16 = sc_v[pl.ds(o, 16)]
                m_gt = k16 > vthr
                m_eq = k16 == vthr
                eq_i = jnp.where(m_eq, 1, 0)
                eq_pre = _prefix16(eq_i, wt_v, iota)
                eq_excl = eq_pre - eq_i
                take_eq = m_eq & ((cnt_eq + eq_excl) < (kk - c1))
                sel = m_gt | take_eq
                sel_i = jnp.where(sel, 1, 0)
                sel_pre = _prefix16(sel_i, wt_v, iota)
                rank = cnt_sel + sel_pre - sel_i
                node16 = o + iota
                del node16
                inv_v[pl.ds(o, 16)] = jnp.where(sel, rank, -1)
                return (cnt_sel + _bcast_sum(sel_i, wt_v, iota),
                        cnt_eq + _bcast_sum(eq_i, wt_v, iota))

            lax.fori_loop(0, nv, sel_body,
                          (jnp.zeros((16,), I32), jnp.zeros((16,), I32)))
            pltpu.sync_copy(inv_v, inv_hbm)
            pltpu.sync_copy(sc_v, score_hbm)

    return k


def _sc_gather_remap(n_p4, kk, k_pad, e4_pad):
    npc = n_p4 // (32 * 128)
    eb = e4_pad // 32

    @functools.partial(
        pl.kernel,
        mesh=_mesh(),
        out_type=(jax.ShapeDtypeStruct((k_pad + 16, DL), F32),
                  jax.ShapeDtypeStruct((k_pad + 16,), F32),
                  jax.ShapeDtypeStruct((e4_pad,), I32),
                  jax.ShapeDtypeStruct((e4_pad,), I32),
                  jax.ShapeDtypeStruct((e4_pad,), F32)),
        scratch_types=[
            pltpu.VMEM((128,), I32),
            pltpu.VMEM((128,), F32),
            pltpu.VMEM((128,), I32),
            pltpu.VMEM((128, DL), F32),
            pltpu.VMEM((CE,), I32),
            pltpu.VMEM((CE,), I32),
            pltpu.VMEM((CE,), F32),
            pltpu.VMEM((CE,), I32),
            pltpu.VMEM((CE,), I32),
            pltpu.VMEM((CE,), I32),
            pltpu.VMEM((CE,), I32),
            pltpu.VMEM((CE,), F32),
            pltpu.SemaphoreType.DMA,
            pltpu.SemaphoreType.DMA,
            pltpu.SemaphoreType.DMA,
        ],
    )
    def k(hh_hbm, inv_hbm, score_hbm, s_hbm, d_hbm, m_hbm,
          nx_hbm, valsh_hbm, ns_hbm, nd_hbm, nm_hbm,
          iv_v, sv_v, ri_v, rows_v, si_v, di_v, m_v, ig_s, ig_d,
          nsv, ndv, nmv, sem1, sem2, sem3):
        c = lax.axis_index("c")
        s = lax.axis_index("s")
        t = s * NCORE + c
        iota = _iota16()

        def node_chunk(j, _):
            n0 = (t * npc + j) * 128
            pltpu.sync_copy(inv_hbm.at[pl.ds(n0, 128)], iv_v)
            pltpu.sync_copy(score_hbm.at[pl.ds(n0, 128)], sv_v)
            pltpu.sync_copy(hh_hbm.at[pl.ds(n0, 128)], rows_v)
            for v in range(8):
                iv16 = iv_v[pl.ds(v * 16, 16)]
                trash = k_pad + iota
                ri_v[pl.ds(v * 16, 16)] = jnp.where(iv16 >= 0, iv16, trash)
            pltpu.async_copy(rows_v, nx_hbm.at[ri_v], sem1).wait()
            pltpu.sync_copy(sv_v, valsh_hbm.at[ri_v])
            return 0

        lax.fori_loop(0, npc, node_chunk, 0)

        e_base = t * eb

        def chunk_body(j, _):
            e0 = e_base + j * CE
            cpa = pltpu.async_copy(s_hbm.at[pl.ds(e0, CE)], si_v, sem1)
            cpb = pltpu.async_copy(d_hbm.at[pl.ds(e0, CE)], di_v, sem2)
            cpc = pltpu.async_copy(m_hbm.at[pl.ds(e0, CE)], m_v, sem3)
            cpa.wait()
            cp1 = pltpu.async_copy(inv_hbm.at[si_v], ig_s, sem1)
            cpb.wait()
            cp2 = pltpu.async_copy(inv_hbm.at[di_v], ig_d, sem2)
            cpc.wait()
            cp1.wait()
            cp2.wait()
            for v in range(CE // 16):
                o = v * 16
                is16 = ig_s[pl.ds(o, 16)]
                id16 = ig_d[pl.ds(o, 16)]
                m16 = m_v[pl.ds(o, 16)]
                valid = (is16 >= 0) & (id16 >= 0) & (m16 > 0.0)
                sp16 = (e0 + o + iota) % kk
                nsv[pl.ds(o, 16)] = jnp.where(valid, is16, sp16)
                ndv[pl.ds(o, 16)] = jnp.where(valid, id16, sp16)
                nmv[pl.ds(o, 16)] = jnp.where(valid, 1.0, 0.0)
            pltpu.sync_copy(nsv, ns_hbm.at[pl.ds(e0, CE)])
            pltpu.sync_copy(ndv, nd_hbm.at[pl.ds(e0, CE)])
            pltpu.sync_copy(nmv, nm_hbm.at[pl.ds(e0, CE)])
            return 0

        lax.fori_loop(0, eb // CE, chunk_body, 0)

    return k


def kernel(x, edge_index, edge_attr, batch, random_walk_pe, node_attr,
           Wl, Wr, att_w, bconv, prel, proot, pbias):
    del edge_attr, batch, node_attr
    e4_pad = _ceil_to(E, NCORE * NSUB * CE)

    src = edge_index[0]
    dst = edge_index[1]
    pad4 = e4_pad - E
    sp4 = (jnp.arange(pad4, dtype=I32) % jnp.int32(N0))
    s4 = jnp.concatenate([src, sp4])
    d4 = jnp.concatenate([dst, sp4])
    m4 = jnp.concatenate([jnp.ones((E,), F32), jnp.zeros((pad4,), F32)])

    cur = x
    rw = random_walk_pe
    tot = jnp.zeros((1, DL), F32)
    n = N0
    for i in range(4):
        kk = n // 2
        ncalls = 1
        n_p = _ceil_to(n, 128)
        split_edges = i > 0
        nh = n if split_edges else n // 2
        nh_p = _ceil_to(nh, 128)
        ce = 32
        k_pad = _ceil_to(kk, 8)

        xl, xr = _tc_mm2(cur, Wl[i], Wr[i])

        et = E + n
        et_pad = _ceil_to(et, (NCORE if split_edges else 1) * NSUB * ce)
        pad2 = et_pad - et
        sl = jnp.arange(n, dtype=I32)
        sp2 = jnp.arange(pad2, dtype=I32) % jnp.int32(n)
        s2 = jnp.concatenate([s4[:E], sl, sp2])
        d2 = jnp.concatenate([d4[:E], sl, sp2])
        m2 = jnp.concatenate([m4[:E], jnp.ones((n,), F32),
                              jnp.zeros((pad2,), F32)])
        att_flat = att_w[i].reshape(DL)
        zro = jnp.zeros((2 * (nh_p + 16), 128), F32)
        del ncalls
        af0, af1, d00, d10, d01, d11 = _sc_edge_attention(
            nh, 0, nh_p, et_pad, ce, split_edges)(
            xl, xr, s2, d2, m2, att_flat, zro)
        if split_edges:
            acc = (af0 + af1).reshape(nh_p + 16, DL)[:n]
            den0 = (d00 + d01)[:n]
            den1 = (d10 + d11)[:n]
        else:
            acc = jnp.concatenate(
                [af0.reshape(nh_p + 16, DL)[:nh],
                 af1.reshape(nh_p + 16, DL)[:nh]], axis=0)
            den0 = jnp.concatenate([d00[:nh], d01[:nh]])
            den1 = jnp.concatenate([d10[:nh], d11[:nh]])

        hh, a, bsc = _tc_post(acc, den0, den1, rw, bconv[i], prel[i],
                              proot[i], pbias[i])

        a_pad = jnp.concatenate([a.reshape(n), jnp.zeros((n_p - n,), F32)])
        nb0, nb1 = _sc_scorer(n_p, e4_pad)(a_pad, s4, d4, m4)

        bsc_pad = jnp.concatenate([bsc.reshape(n),
                                   jnp.full((n_p - n,), -1e30, F32)])
        inv, score = _sc_select(n_p, kk)(nb0, nb1, bsc_pad)

        n_p4 = _ceil_to(n_p, 32 * 128)
        hh4 = jnp.concatenate([hh, jnp.zeros((n_p4 - n, DL), F32)], axis=0)
        inv4 = jnp.concatenate([inv, jnp.full((n_p4 - n_p,), -1, I32)])
        score4 = jnp.concatenate([score, jnp.zeros((n_p4 - n_p,), F32)])
        nxt, valsh, ns, nd, nm = _sc_gather_remap(n_p4, kk, k_pad, e4_pad)(
            hh4, inv4, score4, s4, d4, m4)

        tot, cur = _tc_finish(tot, nxt[:kk], valsh[:kk].reshape(kk, 1))

        rw = cur[:, HID:HID + WL]
        s4, d4, m4 = ns, nd, nm
        n = kk

    return tot


# speedup vs baseline: 8.4713x; 1.0005x over previous
import functools

import jax
import jax.numpy as jnp
from jax import lax
from jax.experimental import pallas as pl
from jax.experimental.pallas import tpu as pltpu
from jax.experimental.pallas import tpu_sc as plsc

N0, E, D_IN, HEADS, HID, WL = 10000, 160000, 144, 2, 128, 16
DL = 2 * HID
DACC = DL + 128
CE = 128
NSUB = 16
NCORE = 2
F32 = jnp.float32
I32 = jnp.int32


def _ceil_to(x, m):
    return m * ((x + m - 1) // m)


def _mm2_body(x_ref, wl_ref, wr_ref, xl_ref, xr_ref):
    x = x_ref[...]
    xl_ref[...] = jnp.dot(x, wl_ref[...], preferred_element_type=F32)
    xr_ref[...] = jnp.dot(x, wr_ref[...], preferred_element_type=F32)


def _tc_mm2(cur, wl, wr):
    n = cur.shape[0]
    return pl.pallas_call(
        _mm2_body,
        out_shape=(jax.ShapeDtypeStruct((n, DL), F32),
                   jax.ShapeDtypeStruct((n, DL), F32)),
    )(cur, wl, wr)


def _post_body(acc_ref, d0_ref, d1_ref, rw_ref, b_ref, prel_ref, proot_ref,
               pb_ref, hh_ref, a_ref, bsc_ref):
    acc = acc_ref[...]
    o0 = acc[:, 0:HID]
    o1 = acc[:, HID:DL]
    d0 = d0_ref[...]
    d1 = d1_ref[...]
    h = jax.nn.relu((o0 / (d0 + 1e-16) + o1 / (d1 + 1e-16)) * 0.5 + b_ref[...])
    hh_ref[:, 0:HID] = h
    hh_ref[:, HID:D_IN] = rw_ref[...]
    hh_ref[:, D_IN:DL] = jnp.zeros((h.shape[0], DL - D_IN), F32)
    a_ref[...] = jnp.dot(h, prel_ref[...], preferred_element_type=F32)
    bsc_ref[...] = jnp.dot(h, proot_ref[...], preferred_element_type=F32) + pb_ref[...]


def _tc_post(acc, den0, den1, rw, bconv, prel, proot, pb):
    n = acc.shape[0]
    return pl.pallas_call(
        _post_body,
        out_shape=(jax.ShapeDtypeStruct((n, DL), F32),
                   jax.ShapeDtypeStruct((n, 1), F32),
                   jax.ShapeDtypeStruct((n, 1), F32)),
    )(acc, den0.reshape(n, 1), den1.reshape(n, 1), rw.reshape(n, WL),
      bconv.reshape(1, HID), prel, proot, pb.reshape(1, 1))


def _finish_body(tot_ref, nxt_ref, vals_ref, tot_out_ref, cur_ref):
    sc = jnp.tanh(vals_ref[...])
    nx = nxt_ref[:, 0:HID] * sc
    mean = jnp.mean(nx, axis=0)
    mx = jnp.max(nx, axis=0)
    tot_out_ref[...] = tot_ref[...] + jnp.concatenate([mean, mx]).reshape(1, DL)
    cur_ref[:, 0:HID] = nx
    cur_ref[:, HID:D_IN] = nxt_ref[:, HID:D_IN]


def _tc_finish(tot, nxt, vals):
    kk = nxt.shape[0]
    return pl.pallas_call(
        _finish_body,
        out_shape=(jax.ShapeDtypeStruct((1, DL), F32),
                   jax.ShapeDtypeStruct((kk, D_IN), F32)),
    )(tot, nxt, vals)


def _mesh():
    return plsc.VectorSubcoreMesh(core_axis_name="c", subcore_axis_name="s")


def _iota16():
    return lax.iota(I32, 16)


def _prefix16(v, wtmp, iota):
    del iota
    x = v
    for k in (1, 2, 4, 8):
        wtmp[pl.ds(16, 16)] = x
        x = x + wtmp[pl.ds(16 - k, 16)]
    return x


def _bcast_sum(v, wtmp, iota):
    x = v
    for k in (8, 4, 2, 1):
        wtmp[pl.ds(16, 16)] = x
        left = wtmp[pl.ds(16 + k, 16)]
        right = wtmp[pl.ds(16 - k, 16)]
        x = x + jnp.where((iota & k) == 0, left, right)
    return x


def _sc_edge_attention(nh, base0, nh_p, et_pad, ce, split_edges):
    cpt = et_pad // ((NCORE if split_edges else 1) * NSUB * ce)
    nrow = nh_p + 16
    flat = nrow * DL

    @functools.partial(
        pl.kernel,
        mesh=_mesh(),
        out_type=(jax.ShapeDtypeStruct((2 * nrow, 128), F32),
                  jax.ShapeDtypeStruct((2 * nrow, 128), F32),
                  jax.ShapeDtypeStruct((nrow,), F32),
                  jax.ShapeDtypeStruct((nrow,), F32),
                  jax.ShapeDtypeStruct((nrow,), F32),
                  jax.ShapeDtypeStruct((nrow,), F32)),
        scratch_types=[
            pltpu.VMEM((ce,), I32),
            pltpu.VMEM((ce,), I32),
            pltpu.VMEM((ce,), F32),
            pltpu.VMEM((ce,), I32),
            pltpu.VMEM((ce,), I32),
            pltpu.VMEM((ce,), I32),
            pltpu.VMEM((ce,), F32),
            pltpu.VMEM((ce,), F32),
            pltpu.VMEM((ce, DL), F32),
            pltpu.VMEM((ce, DL), F32),
            pltpu.VMEM((ce, 128), F32),
            pltpu.VMEM((ce, 128), F32),
            pltpu.VMEM((DL,), F32),
            pltpu.VMEM((48,), F32),
            pltpu.VMEM((48,), F32),
            pltpu.VMEM((nrow,), F32),
            pltpu.VMEM_SHARED((2 * nrow, 128), F32),
            pltpu.VMEM_SHARED((nrow,), F32),
            pltpu.VMEM_SHARED((nrow,), F32),
            pltpu.SemaphoreType.DMA,
            pltpu.SemaphoreType.DMA,
        ],
    )
    def k(xl_hbm, xr_hbm, s_hbm, d_hbm, m_hbm, att_hbm, zro_hbm,
          accf_hbm0, accf_hbm1, d0o_hbm0, d1o_hbm0, d0o_hbm1, d1o_hbm1,
          si_v, di_v, m_v, ri_v, idx0_v, idx1_v, w0c_v, w1c_v, xls_v, xrd_v,
          oe_v, oo_v, att_v, wtf_v, wtg_v, zd_v,
          acc_sp, den0_sp, den1_sp, sem1, sem2):
        c = lax.axis_index("c")
        s = lax.axis_index("s")
        base = 0 if split_edges else base0 + c * nh
        iota = _iota16()

        pltpu.sync_copy(att_hbm, att_v)
        for j3 in range(3):
            wtf_v[pl.ds(j3 * 16, 16)] = jnp.zeros((16,), F32)
            wtg_v[pl.ds(j3 * 16, 16)] = jnp.zeros((16,), F32)

        @pl.when(s == 0)
        def _():
            pltpu.sync_copy(zro_hbm, acc_sp)

            def zb(i, _):
                zd_v[pl.ds(i * 16, 16)] = jnp.zeros((16,), F32)
                return 0
            lax.fori_loop(0, nrow // 16, zb, 0)
            pltpu.sync_copy(zd_v, den0_sp)
            pltpu.sync_copy(zd_v, den1_sp)

        plsc.subcore_barrier()

        def chunk_body(j, _):
            if split_edges:
                cid = (c * NSUB + s) * cpt + j
            else:
                cid = s * cpt + j
            e0 = cid * ce
            cpa = pltpu.async_copy(s_hbm.at[pl.ds(e0, ce)], si_v, sem1)
            cpb = pltpu.async_copy(d_hbm.at[pl.ds(e0, ce)], di_v, sem2)
            pltpu.sync_copy(m_hbm.at[pl.ds(e0, ce)], m_v)
            cpa.wait()
            cpb.wait()
            for v in range(ce // 16):
                d16 = di_v[pl.ds(v * 16, 16)]
                m16 = m_v[pl.ds(v * 16, 16)]
                if split_edges:
                    live = m16 > 0.0
                else:
                    live = (d16 >= base) & (d16 < base + nh) & (m16 > 0.0)
                trash = nh_p + (iota % 8) + (v % 2) * 8
                r16 = jnp.where(live, d16 - base, trash)
                ri_v[pl.ds(v * 16, 16)] = r16
                idx0_v[pl.ds(v * 16, 16)] = r16 * 2
                idx1_v[pl.ds(v * 16, 16)] = r16 * 2 + 1
                w0c_v[pl.ds(v * 16, 16)] = jnp.zeros((16,), F32)
                w1c_v[pl.ds(v * 16, 16)] = jnp.zeros((16,), F32)
            cp1 = pltpu.async_copy(xl_hbm.at[si_v], xls_v, sem1)
            cp2 = pltpu.async_copy(xr_hbm.at[di_v], xrd_v, sem2)
            cp1.wait()
            cp2.wait()

            def edge_body(e, _):
                xlr = xls_v.at[e]
                xrr = xrd_v.at[e]
                accs = []
                for hh in range(HEADS):
                    accv = jnp.zeros((16,), F32)
                    for f in range(HID // 16):
                        c0 = hh * HID + f * 16
                        v1 = xlr[pl.ds(c0, 16)] + xrr[pl.ds(c0, 16)]
                        v1 = jnp.maximum(v1, 0.2 * v1)
                        accv = accv + v1 * att_v[pl.ds(c0, 16)]
                    accs.append(accv)
                w0 = jnp.exp(_bcast_sum(accs[0], wtf_v, iota))
                w1 = jnp.exp(_bcast_sum(accs[1], wtg_v, iota))
                vb = (e // 16) * 16
                lane = e - vb
                oh = iota == lane
                ohf = jnp.where(oh, 1.0, 0.0)
                t0 = w0c_v[pl.ds(vb, 16)]
                w0c_v[pl.ds(vb, 16)] = t0 + w0 * ohf
                t1 = w1c_v[pl.ds(vb, 16)]
                w1c_v[pl.ds(vb, 16)] = t1 + w1 * ohf
                o0 = oe_v.at[e]
                o1 = oo_v.at[e]
                for f in range(HID // 16):
                    c0 = f * 16
                    o0[pl.ds(c0, 16)] = w0 * xlr[pl.ds(c0, 16)]
                    o1[pl.ds(c0, 16)] = w1 * xlr[pl.ds(HID + c0, 16)]
                return 0

            lax.fori_loop(0, ce, edge_body, 0)
            pltpu.sync_copy(w0c_v, den0_sp.at[ri_v], add=True)
            pltpu.sync_copy(w1c_v, den1_sp.at[ri_v], add=True)
            pltpu.sync_copy(oe_v, acc_sp.at[idx0_v], add=True)
            pltpu.sync_copy(oo_v, acc_sp.at[idx1_v], add=True)
            return 0

        lax.fori_loop(0, cpt, chunk_body, 0)
        plsc.subcore_barrier()

        @pl.when((s == 0) & (c == 0))
        def _():
            pltpu.sync_copy(acc_sp, accf_hbm0)
            pltpu.sync_copy(den0_sp, d0o_hbm0)
            pltpu.sync_copy(den1_sp, d1o_hbm0)

        @pl.when((s == 0) & (c == 1))
        def _():
            pltpu.sync_copy(acc_sp, accf_hbm1)
            pltpu.sync_copy(den0_sp, d0o_hbm1)
            pltpu.sync_copy(den1_sp, d1o_hbm1)

    return k


def _sc_scorer(n_p, e4_pad):
    cpt = e4_pad // (NCORE * NSUB * CE)

    @functools.partial(
        pl.kernel,
        mesh=_mesh(),
        out_type=(jax.ShapeDtypeStruct((n_p,), F32),
                  jax.ShapeDtypeStruct((n_p,), F32)),
        scratch_types=[
            pltpu.VMEM((n_p,), F32),
            pltpu.VMEM((CE,), I32),
            pltpu.VMEM((CE,), I32),
            pltpu.VMEM((CE,), F32),
            pltpu.VMEM((CE,), F32),
            pltpu.VMEM((CE,), F32),
            pltpu.VMEM_SHARED((n_p,), F32),
            pltpu.SemaphoreType.DMA,
            pltpu.SemaphoreType.DMA,
            pltpu.SemaphoreType.DMA,
        ],
    )
    def k(a_hbm, s_hbm, d_hbm, m_hbm, nb0_hbm, nb1_hbm,
          z_v, si_v, di_v, m_v, ag_v, up_v, nb_sp, sem, sem2, sem3):
        c = lax.axis_index("c")
        s = lax.axis_index("s")

        @pl.when(s == 0)
        def _():
            def zb(i, _):
                z_v[pl.ds(i * 16, 16)] = jnp.zeros((16,), F32)
                return 0
            lax.fori_loop(0, n_p // 16, zb, 0)
            pltpu.sync_copy(z_v, nb_sp)

        plsc.subcore_barrier()

        def chunk_body(j, _):
            cid = (c * NSUB + s) * cpt + j
            e0 = cid * CE
            cp1 = pltpu.async_copy(s_hbm.at[pl.ds(e0, CE)], si_v, sem)
            cp2 = pltpu.async_copy(d_hbm.at[pl.ds(e0, CE)], di_v, sem2)
            cp3 = pltpu.async_copy(m_hbm.at[pl.ds(e0, CE)], m_v, sem3)
            cp1.wait()
            cpg = pltpu.async_copy(a_hbm.at[si_v], ag_v, sem)
            cp2.wait()
            cp3.wait()
            cpg.wait()
            for v in range(CE // 16):
                a16 = ag_v[pl.ds(v * 16, 16)]
                m16 = m_v[pl.ds(v * 16, 16)]
                up_v[pl.ds(v * 16, 16)] = a16 * m16
            pltpu.sync_copy(up_v, nb_sp.at[di_v], add=True)
            return 0

        lax.fori_loop(0, cpt, chunk_body, 0)
        plsc.subcore_barrier()

        @pl.when((s == 0) & (c == 0))
        def _():
            pltpu.sync_copy(nb_sp, nb0_hbm)

        @pl.when((s == 0) & (c == 1))
        def _():
            pltpu.sync_copy(nb_sp, nb1_hbm)

    return k


def _sc_select(n_p, kk):
    nv = n_p // 16

    @functools.partial(
        pl.kernel,
        mesh=_mesh(),
        out_type=(jax.ShapeDtypeStruct((n_p,), I32),
                  jax.ShapeDtypeStruct((n_p,), F32)),
        scratch_types=[
            pltpu.VMEM((n_p,), F32),
            pltpu.VMEM((n_p,), jnp.uint32),
            pltpu.VMEM((n_p,), F32),
            pltpu.VMEM((n_p,), F32),
            pltpu.VMEM((n_p,), F32),
            pltpu.VMEM((n_p,), I32),
            pltpu.VMEM((48,), I32),
        ],
    )
    def k(nb0_hbm, nb1_hbm, bsc_hbm, inv_hbm, score_hbm,
          sc_v, key_v, nb0_v, nb1_v, bsc_v, inv_v, wt_v):
        c = lax.axis_index("c")
        s = lax.axis_index("s")

        @pl.when((c == 0) & (s == 0))
        def _():
            pltpu.sync_copy(nb0_hbm, nb0_v)
            pltpu.sync_copy(nb1_hbm, nb1_v)
            pltpu.sync_copy(bsc_hbm, bsc_v)
            iota = _iota16()
            for j3 in range(3):
                wt_v[pl.ds(j3 * 16, 16)] = jnp.zeros((16,), I32)

            def keys_body(v, _):
                o = v * 16
                sc16 = (nb0_v[pl.ds(o, 16)] + nb1_v[pl.ds(o, 16)]
                        + bsc_v[pl.ds(o, 16)])
                sc_v[pl.ds(o, 16)] = sc16
                bu = lax.bitcast_convert_type(sc16, jnp.uint32)
                neg = bu >= jnp.uint32(0x80000000)
                key = jnp.where(neg, ~bu, bu | jnp.uint32(0x80000000))
                key_v[pl.ds(o, 16)] = key
                return 0

            lax.fori_loop(0, nv, keys_body, 0)

            def count_ge(tv):
                def cb(v, acc):
                    k16 = key_v[pl.ds(v * 16, 16)]
                    return acc + jnp.where(k16 >= tv, 1, 0)
                pc = lax.fori_loop(0, nv, cb, jnp.zeros((16,), I32))
                return _bcast_sum(pc, wt_v, iota)

            lo = jnp.zeros((16,), jnp.uint32)
            for bit in range(31, -1, -1):
                cand = lo | jnp.uint32(1 << bit)
                cnt = count_ge(cand)
                lo = jnp.where(cnt >= kk, cand, lo)
            vthr = lo
            c1 = count_ge(vthr + jnp.uint32(1))

            def sel_body(v, carry):
                cnt_sel, cnt_eq = carry
                o = v * 16
                k16 = key_v[pl.ds(o, 16)]
                s16 = sc_v[pl.ds(o, 16)]
                m_gt = k16 > vthr
                m_eq = k16 == vthr
                eq_i = jnp.where(m_eq, 1, 0)
                eq_pre = _prefix16(eq_i, wt_v, iota)
                eq_excl = eq_pre - eq_i
                take_eq = m_eq & ((cnt_eq + eq_excl) < (kk - c1))
                sel = m_gt | take_eq
                sel_i = jnp.where(sel, 1, 0)
                sel_pre = _prefix16(sel_i, wt_v, iota)
                rank = cnt_sel + sel_pre - sel_i
                inv_v[pl.ds(o, 16)] = jnp.where(sel, rank, -1)
                return (cnt_sel + _bcast_sum(sel_i, wt_v, iota),
                        cnt_eq + _bcast_sum(eq_i, wt_v, iota))

            lax.fori_loop(0, nv, sel_body,
                          (jnp.zeros((16,), I32), jnp.zeros((16,), I32)))
            pltpu.sync_copy(inv_v, inv_hbm)
            pltpu.sync_copy(sc_v, score_hbm)

    return k


def _sc_gather_remap(n_p4, kk, k_pad, e4_pad):
    npc = n_p4 // (32 * 128)
    eb = e4_pad // 32

    @functools.partial(
        pl.kernel,
        mesh=_mesh(),
        out_type=(jax.ShapeDtypeStruct((k_pad + 16, DL), F32),
                  jax.ShapeDtypeStruct((k_pad + 16,), F32),
                  jax.ShapeDtypeStruct((e4_pad,), I32),
                  jax.ShapeDtypeStruct((e4_pad,), I32),
                  jax.ShapeDtypeStruct((e4_pad,), F32)),
        scratch_types=[
            pltpu.VMEM((128,), I32),
            pltpu.VMEM((128,), F32),
            pltpu.VMEM((128,), I32),
            pltpu.VMEM((128, DL), F32),
            pltpu.VMEM((CE,), I32),
            pltpu.VMEM((CE,), I32),
            pltpu.VMEM((CE,), F32),
            pltpu.VMEM((CE,), I32),
            pltpu.VMEM((CE,), I32),
            pltpu.VMEM((CE,), I32),
            pltpu.VMEM((CE,), I32),
            pltpu.VMEM((CE,), F32),
            pltpu.SemaphoreType.DMA,
            pltpu.SemaphoreType.DMA,
            pltpu.SemaphoreType.DMA,
        ],
    )
    def k(hh_hbm, inv_hbm, score_hbm, s_hbm, d_hbm, m_hbm,
          nx_hbm, valsh_hbm, ns_hbm, nd_hbm, nm_hbm,
          iv_v, sv_v, ri_v, rows_v, si_v, di_v, m_v, ig_s, ig_d,
          nsv, ndv, nmv, sem1, sem2, sem3):
        c = lax.axis_index("c")
        s = lax.axis_index("s")
        t = s * NCORE + c
        iota = _iota16()

        def node_chunk(j, _):
            n0 = (t * npc + j) * 128
            pltpu.sync_copy(inv_hbm.at[pl.ds(n0, 128)], iv_v)
            pltpu.sync_copy(score_hbm.at[pl.ds(n0, 128)], sv_v)
            pltpu.sync_copy(hh_hbm.at[pl.ds(n0, 128)], rows_v)
            for v in range(8):
                iv16 = iv_v[pl.ds(v * 16, 16)]
                trash = k_pad + iota
                ri_v[pl.ds(v * 16, 16)] = jnp.where(iv16 >= 0, iv16, trash)
            pltpu.async_copy(rows_v, nx_hbm.at[ri_v], sem1).wait()
            pltpu.sync_copy(sv_v, valsh_hbm.at[ri_v])
            return 0

        lax.fori_loop(0, npc, node_chunk, 0)

        e_base = t * eb

        def chunk_body(j, _):
            e0 = e_base + j * CE
            cpa = pltpu.async_copy(s_hbm.at[pl.ds(e0, CE)], si_v, sem1)
            cpb = pltpu.async_copy(d_hbm.at[pl.ds(e0, CE)], di_v, sem2)
            cpc = pltpu.async_copy(m_hbm.at[pl.ds(e0, CE)], m_v, sem3)
            cpa.wait()
            cp1 = pltpu.async_copy(inv_hbm.at[si_v], ig_s, sem1)
            cpb.wait()
            cp2 = pltpu.async_copy(inv_hbm.at[di_v], ig_d, sem2)
            cpc.wait()
            cp1.wait()
            cp2.wait()
            for v in range(CE // 16):
                o = v * 16
                is16 = ig_s[pl.ds(o, 16)]
                id16 = ig_d[pl.ds(o, 16)]
                m16 = m_v[pl.ds(o, 16)]
                valid = (is16 >= 0) & (id16 >= 0) & (m16 > 0.0)
                sp16 = (e0 + o + iota) % kk
                nsv[pl.ds(o, 16)] = jnp.where(valid, is16, sp16)
                ndv[pl.ds(o, 16)] = jnp.where(valid, id16, sp16)
                nmv[pl.ds(o, 16)] = jnp.where(valid, 1.0, 0.0)
            pltpu.sync_copy(nsv, ns_hbm.at[pl.ds(e0, CE)])
            pltpu.sync_copy(ndv, nd_hbm.at[pl.ds(e0, CE)])
            pltpu.sync_copy(nmv, nm_hbm.at[pl.ds(e0, CE)])
            return 0

        lax.fori_loop(0, eb // CE, chunk_body, 0)

    return k


def kernel(x, edge_index, edge_attr, batch, random_walk_pe, node_attr,
           Wl, Wr, att_w, bconv, prel, proot, pbias):
    del edge_attr, batch, node_attr
    e4_pad = _ceil_to(E, NCORE * NSUB * CE)

    src = edge_index[0]
    dst = edge_index[1]
    pad4 = e4_pad - E
    sp4 = (jnp.arange(pad4, dtype=I32) % jnp.int32(N0))
    s4 = jnp.concatenate([src, sp4])
    d4 = jnp.concatenate([dst, sp4])
    m4 = jnp.concatenate([jnp.ones((E,), F32), jnp.zeros((pad4,), F32)])

    cur = x
    rw = random_walk_pe
    tot = jnp.zeros((1, DL), F32)
    n = N0
    for i in range(4):
        kk = n // 2
        n_p = _ceil_to(n, 128)
        split_edges = i > 0
        nh = n if split_edges else n // 2
        nh_p = _ceil_to(nh, 128)
        ce = 32
        k_pad = _ceil_to(kk, 8)

        xl, xr = _tc_mm2(cur, Wl[i], Wr[i])

        et = E + n
        et_pad = _ceil_to(et, (NCORE if split_edges else 1) * NSUB * ce)
        pad2 = et_pad - et
        sl = jnp.arange(n, dtype=I32)
        sp2 = jnp.arange(pad2, dtype=I32) % jnp.int32(n)
        s2 = jnp.concatenate([s4[:E], sl, sp2])
        d2 = jnp.concatenate([d4[:E], sl, sp2])
        m2 = jnp.concatenate([m4[:E], jnp.ones((n,), F32),
                              jnp.zeros((pad2,), F32)])
        att_flat = att_w[i].reshape(DL)
        zro = jnp.zeros((2 * (nh_p + 16), 128), F32)
        af0, af1, d00, d10, d01, d11 = _sc_edge_attention(
            nh, 0, nh_p, et_pad, ce, split_edges)(
            xl, xr, s2, d2, m2, att_flat, zro)
        if split_edges:
            acc = (af0 + af1).reshape(nh_p + 16, DL)[:n]
            den0 = (d00 + d01)[:n]
            den1 = (d10 + d11)[:n]
        else:
            acc = jnp.concatenate(
                [af0.reshape(nh_p + 16, DL)[:nh],
                 af1.reshape(nh_p + 16, DL)[:nh]], axis=0)
            den0 = jnp.concatenate([d00[:nh], d01[:nh]])
            den1 = jnp.concatenate([d10[:nh], d11[:nh]])

        hh, a, bsc = _tc_post(acc, den0, den1, rw, bconv[i], prel[i],
                              proot[i], pbias[i])

        a_pad = jnp.concatenate([a.reshape(n), jnp.zeros((n_p - n,), F32)])
        nb0, nb1 = _sc_scorer(n_p, e4_pad)(a_pad, s4, d4, m4)

        bsc_pad = jnp.concatenate([bsc.reshape(n),
                                   jnp.full((n_p - n,), -1e30, F32)])
        inv, score = _sc_select(n_p, kk)(nb0, nb1, bsc_pad)

        n_p4 = _ceil_to(n_p, 32 * 128)
        hh4 = jnp.concatenate([hh, jnp.zeros((n_p4 - n, DL), F32)], axis=0)
        inv4 = jnp.concatenate([inv, jnp.full((n_p4 - n_p,), -1, I32)])
        score4 = jnp.concatenate([score, jnp.zeros((n_p4 - n_p,), F32)])
        nxt, valsh, ns, nd, nm = _sc_gather_remap(n_p4, kk, k_pad, e4_pad)(
            hh4, inv4, score4, s4, d4, m4)

        tot, cur = _tc_finish(tot, nxt[:kk], valsh[:kk].reshape(kk, 1))

        rw = cur[:, HID:HID + WL]
        s4, d4, m4 = ns, nd, nm
        n = kk

    return tot
